# Initial kernel scaffold; baseline (speedup 1.0000x reference)
import jax
import jax.numpy as jnp
from jax.experimental import pallas as pl


def kernel(des, tweet, num_prop, cat_prop, edge_index, edge_type, W_des, b_des, W_tweet, b_tweet, W_num, b_num, W_cat, b_cat, W_in, b_in, W_rel, W_root, b_rgcn, W_out1, b_out1, W_out2, b_out2):
    raise NotImplementedError("write your pallas kernel here")


# speedup vs baseline: 7.0526x; 7.0526x over previous
import functools

import jax
import jax.numpy as jnp
from jax import lax
from jax.experimental import pallas as pl
from jax.experimental.pallas import tpu as pltpu
from jax.experimental.pallas import tpu_sc as plsc

N = 50000
E = 1600000
D = 32
NUM_REL = 2

NTILES = 16
CHUNK = 512
IDXW = 128
RPC = CHUNK // IDXW
EPAD = -(-E // (NTILES * CHUNK)) * (NTILES * CHUNK)
CH_PER_TILE = EPAD // (NTILES * CHUNK)
N_TAB = 50048
TRASH = 50000
RB = 2000
GRID = N // RB


def _lk(x):
  return jnp.where(x > 0, x, 0.01 * x)


_sc_mesh = plsc.VectorSubcoreMesh(core_axis_name="c", subcore_axis_name="s")


@functools.partial(
    pl.kernel,
    out_type=jax.ShapeDtypeStruct((NUM_REL, N_TAB, D), jnp.float32),
    mesh=_sc_mesh,
    compiler_params=pltpu.CompilerParams(use_tc_tiling_on_sc=False),
    scratch_types=[
        pltpu.VMEM((RPC, IDXW), jnp.int32),
        pltpu.VMEM((RPC, IDXW), jnp.int32),
        pltpu.VMEM((RPC, IDXW), jnp.int32),
        pltpu.VMEM((RPC, IDXW), jnp.int32),
        pltpu.VMEM((CHUNK, D), jnp.float32),
        pltpu.VMEM_SHARED((N_TAB, D), jnp.float32),
        pltpu.SemaphoreType.DMA,
        pltpu.SemaphoreType.DMA,
    ],
)
def _sc_segsum(x_hbm, src_hbm, dst_hbm, typ_hbm, zeros_hbm, out_hbm,
               src_v, dst_v, typ_v, dstp_v, rows_v, table, gsem, ssem):
  c = lax.axis_index("c")
  s = lax.axis_index("s")

  zrows = N_TAB // NTILES
  pltpu.sync_copy(zeros_hbm.at[pl.ds(s * zrows, zrows)],
                  table.at[pl.ds(s * zrows, zrows)])
  plsc.subcore_barrier()

  def body(i, carry):
    row0 = (s * CH_PER_TILE + i) * RPC
    pltpu.sync_copy(src_hbm.at[pl.ds(row0, RPC)], src_v)
    pltpu.sync_copy(dst_hbm.at[pl.ds(row0, RPC)], dst_v)
    pltpu.sync_copy(typ_hbm.at[pl.ds(row0, RPC)], typ_v)
    gcps = [
        pltpu.async_copy(x_hbm.at[src_v.at[j]],
                         rows_v.at[pl.ds(j * IDXW, IDXW)], gsem)
        for j in range(RPC)
    ]
    for j in range(RPC):
      for l in range(IDXW // 16):
        t16 = typ_v[j, pl.ds(l * 16, 16)]
        d16 = dst_v[j, pl.ds(l * 16, 16)]
        dstp_v[j, pl.ds(l * 16, 16)] = jnp.where(t16 == c, d16, TRASH)
    for cp in gcps:
      cp.wait()
    scps = [
        pltpu.async_copy(rows_v.at[pl.ds(j * IDXW, IDXW)],
                         table.at[dstp_v.at[j]], ssem, add=True)
        for j in range(RPC)
    ]
    for cp in scps:
      cp.wait()
    return carry

  lax.fori_loop(0, CH_PER_TILE, body, 0)
  plsc.subcore_barrier()

  wrows = N_TAB // NTILES
  pltpu.sync_copy(table.at[pl.ds(s * wrows, wrows)],
                  out_hbm.at[c, pl.ds(s * wrows, wrows)])


def _enc_body(des_r, tw_r, np_r, cp_r, wd, bd, wt, bt, wn, bn, wc, bc,
              win, bin_, out_r):
  d = _lk(jnp.dot(des_r[...], wd[...], preferred_element_type=jnp.float32)
          + bd[...])
  t = _lk(jnp.dot(tw_r[...], wt[...], preferred_element_type=jnp.float32)
          + bt[...])
  n = _lk(jnp.dot(np_r[...], wn[...], preferred_element_type=jnp.float32)
          + bn[...])
  cc = _lk(jnp.dot(cp_r[...], wc[...], preferred_element_type=jnp.float32)
           + bc[...])
  x = jnp.concatenate([d, t, n, cc], axis=1)
  out_r[...] = _lk(jnp.dot(x, win[...], preferred_element_type=jnp.float32)
                   + bin_[...])


def _full(shape):
  return pl.BlockSpec(shape, lambda i: (0, 0))


def _encoder(des, tweet, num_prop, cat_prop, wd, bd, wt, bt, wn, bn, wc, bc,
             win, bin_):
  return pl.pallas_call(
      _enc_body,
      grid=(GRID,),
      in_specs=[
          pl.BlockSpec((RB, 768), lambda i: (i, 0)),
          pl.BlockSpec((RB, 768), lambda i: (i, 0)),
          pl.BlockSpec((RB, 6), lambda i: (i, 0)),
          pl.BlockSpec((RB, 3), lambda i: (i, 0)),
          _full((768, D // 4)), _full((1, D // 4)),
          _full((768, D // 4)), _full((1, D // 4)),
          _full((6, D // 4)), _full((1, D // 4)),
          _full((3, D // 4)), _full((1, D // 4)),
          _full((D, D)), _full((1, D)),
      ],
      out_specs=pl.BlockSpec((RB, D), lambda i: (i, 0)),
      out_shape=jax.ShapeDtypeStruct((N, D), jnp.float32),
  )(des, tweet, num_prop, cat_prop, wd, bd, wt, bt, wn, bn, wc, bc, win, bin_)


def _comb_body(x_r, s0_r, s1_r, c0_r, c1_r, wr, w0, w1, b, out_r):
  m0 = s0_r[...] / jnp.maximum(c0_r[...], 1.0)
  m1 = s1_r[...] / jnp.maximum(c1_r[...], 1.0)
  out_r[...] = (jnp.dot(x_r[...], wr[...], preferred_element_type=jnp.float32)
                + b[...]
                + jnp.dot(m0, w0[...], preferred_element_type=jnp.float32)
                + jnp.dot(m1, w1[...], preferred_element_type=jnp.float32))


def _combine(x, s0, s1, c0, c1, wr, w0, w1, b):
  blk = pl.BlockSpec((RB, D), lambda i: (i, 0))
  return pl.pallas_call(
      _comb_body,
      grid=(GRID,),
      in_specs=[blk, blk, blk, blk, blk,
                _full((D, D)), _full((D, D)), _full((D, D)), _full((1, D))],
      out_specs=pl.BlockSpec((RB, D), lambda i: (i, 0)),
      out_shape=jax.ShapeDtypeStruct((N, D), jnp.float32),
  )(x, s0, s1, c0, c1, wr, w0, w1, b)


def _comb_mlp_body(x_r, s0_r, s1_r, c0_r, c1_r, wr, w0, w1, b,
                   wo1, bo1, wo2, bo2, out_r):
  m0 = s0_r[...] / jnp.maximum(c0_r[...], 1.0)
  m1 = s1_r[...] / jnp.maximum(c1_r[...], 1.0)
  h = (jnp.dot(x_r[...], wr[...], preferred_element_type=jnp.float32)
       + b[...]
       + jnp.dot(m0, w0[...], preferred_element_type=jnp.float32)
       + jnp.dot(m1, w1[...], preferred_element_type=jnp.float32))
  h = _lk(jnp.dot(h, wo1[...], preferred_element_type=jnp.float32) + bo1[...])
  out_r[...] = (jnp.dot(h, wo2[...], preferred_element_type=jnp.float32)
                + bo2[...])


def _combine_mlp(x, s0, s1, c0, c1, wr, w0, w1, b, wo1, bo1, wo2, bo2):
  blk = pl.BlockSpec((RB, D), lambda i: (i, 0))
  return pl.pallas_call(
      _comb_mlp_body,
      grid=(GRID,),
      in_specs=[blk, blk, blk, blk, blk,
                _full((D, D)), _full((D, D)), _full((D, D)), _full((1, D)),
                _full((D, D)), _full((1, D)), _full((D, 2)), _full((1, 2))],
      out_specs=pl.BlockSpec((RB, 2), lambda i: (i, 0)),
      out_shape=jax.ShapeDtypeStruct((N, 2), jnp.float32),
  )(x, s0, s1, c0, c1, wr, w0, w1, b, wo1, bo1, wo2, bo2)


def kernel(des, tweet, num_prop, cat_prop, edge_index, edge_type,
           W_des, b_des, W_tweet, b_tweet, W_num, b_num, W_cat, b_cat,
           W_in, b_in, W_rel, W_root, b_rgcn, W_out1, b_out1, W_out2, b_out2):
  pad = EPAD - E
  src2d = jnp.pad(edge_index[0], (0, pad)).reshape(-1, IDXW)
  dst2d = jnp.pad(edge_index[1], (0, pad)).reshape(-1, IDXW)
  typ2d = jnp.pad(edge_type, (0, pad), constant_values=-1).reshape(-1, IDXW)
  zeros = jnp.zeros((N_TAB, D), jnp.float32)
  ones_x = jnp.ones((N, D), jnp.float32)

  bd = b_des.reshape(1, -1)
  bt = b_tweet.reshape(1, -1)
  bn = b_num.reshape(1, -1)
  bc = b_cat.reshape(1, -1)
  bi = b_in.reshape(1, -1)
  br = b_rgcn.reshape(1, -1)
  bo1 = b_out1.reshape(1, -1)
  bo2 = b_out2.reshape(1, -1)

  x0 = _encoder(des, tweet, num_prop, cat_prop,
                W_des, bd, W_tweet, bt, W_num, bn, W_cat, bc, W_in, bi)

  cnt = _sc_segsum(ones_x, src2d, dst2d, typ2d, zeros)[:, :N]
  s_l1 = _sc_segsum(x0, src2d, dst2d, typ2d, zeros)[:, :N]
  x1 = _combine(x0, s_l1[0], s_l1[1], cnt[0], cnt[1],
                W_root, W_rel[0], W_rel[1], br)
  s_l2 = _sc_segsum(x1, src2d, dst2d, typ2d, zeros)[:, :N]
  out = _combine_mlp(x1, s_l2[0], s_l2[1], cnt[0], cnt[1],
                     W_root, W_rel[0], W_rel[1], br,
                     W_out1, bo1, W_out2, bo2)
  return out


# speedup vs baseline: 7.3060x; 1.0359x over previous
import functools

import jax
import jax.numpy as jnp
from jax import lax
from jax.experimental import pallas as pl
from jax.experimental.pallas import tpu as pltpu
from jax.experimental.pallas import tpu_sc as plsc

N = 50000
E = 1600000
D = 32
NUM_REL = 2

NTILES = 16
CHUNK = 512
IDXW = 128
RPC = CHUNK // IDXW
EPAD = -(-E // (NTILES * CHUNK)) * (NTILES * CHUNK)
CH_PER_TILE = EPAD // (NTILES * CHUNK)
N_TAB = 50048
TRASH = 50000
RB = 2000
GRID = N // RB


def _lk(x):
  return jnp.where(x > 0, x, 0.01 * x)


_sc_mesh = plsc.VectorSubcoreMesh(core_axis_name="c", subcore_axis_name="s")


@functools.partial(
    pl.kernel,
    out_type=jax.ShapeDtypeStruct((NUM_REL, N_TAB, D), jnp.float32),
    mesh=_sc_mesh,
    compiler_params=pltpu.CompilerParams(use_tc_tiling_on_sc=False),
    scratch_types=[
        pltpu.VMEM((RPC, IDXW), jnp.int32),
        pltpu.VMEM((RPC, IDXW), jnp.int32),
        pltpu.VMEM((RPC, IDXW), jnp.int32),
        pltpu.VMEM((RPC, IDXW), jnp.int32),
        pltpu.VMEM((CHUNK, D), jnp.float32),
        pltpu.VMEM_SHARED((N_TAB, D), jnp.float32),
        pltpu.SemaphoreType.DMA,
        pltpu.SemaphoreType.DMA,
    ],
)
def _sc_segsum(x_hbm, src_hbm, dst_hbm, typ_hbm, zeros_hbm, out_hbm,
               src_v, dst_v, typ_v, dstp_v, rows_v, table, gsem, ssem):
  c = lax.axis_index("c")
  s = lax.axis_index("s")

  zrows = N_TAB // NTILES
  pltpu.sync_copy(zeros_hbm.at[pl.ds(s * zrows, zrows)],
                  table.at[pl.ds(s * zrows, zrows)])
  plsc.subcore_barrier()

  def body(i, carry):
    row0 = (s * CH_PER_TILE + i) * RPC
    pltpu.sync_copy(src_hbm.at[pl.ds(row0, RPC)], src_v)
    pltpu.sync_copy(dst_hbm.at[pl.ds(row0, RPC)], dst_v)
    pltpu.sync_copy(typ_hbm.at[pl.ds(row0, RPC)], typ_v)
    gcps = [
        pltpu.async_copy(x_hbm.at[src_v.at[j]],
                         rows_v.at[pl.ds(j * IDXW, IDXW)], gsem)
        for j in range(RPC)
    ]
    for j in range(RPC):
      for l in range(IDXW // 16):
        t16 = typ_v[j, pl.ds(l * 16, 16)]
        d16 = dst_v[j, pl.ds(l * 16, 16)]
        dstp_v[j, pl.ds(l * 16, 16)] = jnp.where(t16 == c, d16, TRASH)
    for cp in gcps:
      cp.wait()
    scps = [
        pltpu.async_copy(rows_v.at[pl.ds(j * IDXW, IDXW)],
                         table.at[dstp_v.at[j]], ssem, add=True)
        for j in range(RPC)
    ]
    for cp in scps:
      cp.wait()
    return carry

  lax.fori_loop(0, CH_PER_TILE, body, 0)
  plsc.subcore_barrier()

  wrows = N_TAB // NTILES
  pltpu.sync_copy(table.at[pl.ds(s * wrows, wrows)],
                  out_hbm.at[c, pl.ds(s * wrows, wrows)])


CW = 8
CCHUNK = 2048
CRPC = CCHUNK // IDXW
CCH_PER_TILE = EPAD // (NTILES * CCHUNK)


@functools.partial(
    pl.kernel,
    out_type=jax.ShapeDtypeStruct((NUM_REL, N_TAB, CW), jnp.float32),
    mesh=_sc_mesh,
    compiler_params=pltpu.CompilerParams(use_tc_tiling_on_sc=False),
    scratch_types=[
        pltpu.VMEM((CRPC, IDXW), jnp.int32),
        pltpu.VMEM((CRPC, IDXW), jnp.int32),
        pltpu.VMEM((CRPC, IDXW), jnp.int32),
        pltpu.VMEM((IDXW, CW), jnp.float32),
        pltpu.VMEM_SHARED((N_TAB, CW), jnp.float32),
        pltpu.SemaphoreType.DMA,
    ],
)
def _sc_count(dst_hbm, typ_hbm, zeros_hbm, ones_hbm, out_hbm,
              dst_v, typ_v, dstp_v, ones_v, table, ssem):
  c = lax.axis_index("c")
  s = lax.axis_index("s")

  zrows = N_TAB // NTILES
  pltpu.sync_copy(zeros_hbm.at[pl.ds(s * zrows, zrows)],
                  table.at[pl.ds(s * zrows, zrows)])
  pltpu.sync_copy(ones_hbm, ones_v)
  plsc.subcore_barrier()

  def body(i, carry):
    row0 = (s * CCH_PER_TILE + i) * CRPC
    pltpu.sync_copy(dst_hbm.at[pl.ds(row0, CRPC)], dst_v)
    pltpu.sync_copy(typ_hbm.at[pl.ds(row0, CRPC)], typ_v)
    for j in range(CRPC):
      for l in range(IDXW // 16):
        t16 = typ_v[j, pl.ds(l * 16, 16)]
        d16 = dst_v[j, pl.ds(l * 16, 16)]
        dstp_v[j, pl.ds(l * 16, 16)] = jnp.where(t16 == c, d16, TRASH)
    scps = [
        pltpu.async_copy(ones_v, table.at[dstp_v.at[j]], ssem, add=True)
        for j in range(CRPC)
    ]
    for cp in scps:
      cp.wait()
    return carry

  lax.fori_loop(0, CCH_PER_TILE, body, 0)
  plsc.subcore_barrier()

  wrows = N_TAB // NTILES
  pltpu.sync_copy(table.at[pl.ds(s * wrows, wrows)],
                  out_hbm.at[c, pl.ds(s * wrows, wrows)])


def _enc_body(des_r, tw_r, np_r, cp_r, wd, bd, wt, bt, wn, bn, wc, bc,
              win, bin_, out_r):
  d = _lk(jnp.dot(des_r[...], wd[...], preferred_element_type=jnp.float32)
          + bd[...])
  t = _lk(jnp.dot(tw_r[...], wt[...], preferred_element_type=jnp.float32)
          + bt[...])
  n = _lk(jnp.dot(np_r[...], wn[...], preferred_element_type=jnp.float32)
          + bn[...])
  cc = _lk(jnp.dot(cp_r[...], wc[...], preferred_element_type=jnp.float32)
           + bc[...])
  x = jnp.concatenate([d, t, n, cc], axis=1)
  out_r[...] = _lk(jnp.dot(x, win[...], preferred_element_type=jnp.float32)
                   + bin_[...])


def _full(shape):
  return pl.BlockSpec(shape, lambda i: (0, 0))


def _encoder(des, tweet, num_prop, cat_prop, wd, bd, wt, bt, wn, bn, wc, bc,
             win, bin_):
  return pl.pallas_call(
      _enc_body,
      grid=(GRID,),
      in_specs=[
          pl.BlockSpec((RB, 768), lambda i: (i, 0)),
          pl.BlockSpec((RB, 768), lambda i: (i, 0)),
          pl.BlockSpec((RB, 6), lambda i: (i, 0)),
          pl.BlockSpec((RB, 3), lambda i: (i, 0)),
          _full((768, D // 4)), _full((1, D // 4)),
          _full((768, D // 4)), _full((1, D // 4)),
          _full((6, D // 4)), _full((1, D // 4)),
          _full((3, D // 4)), _full((1, D // 4)),
          _full((D, D)), _full((1, D)),
      ],
      out_specs=pl.BlockSpec((RB, D), lambda i: (i, 0)),
      out_shape=jax.ShapeDtypeStruct((N, D), jnp.float32),
  )(des, tweet, num_prop, cat_prop, wd, bd, wt, bt, wn, bn, wc, bc, win, bin_)


def _comb_body(x_r, s0_r, s1_r, c0_r, c1_r, wr, w0, w1, b, out_r):
  m0 = s0_r[...] / jnp.maximum(c0_r[...][:, :1], 1.0)
  m1 = s1_r[...] / jnp.maximum(c1_r[...][:, :1], 1.0)
  out_r[...] = (jnp.dot(x_r[...], wr[...], preferred_element_type=jnp.float32)
                + b[...]
                + jnp.dot(m0, w0[...], preferred_element_type=jnp.float32)
                + jnp.dot(m1, w1[...], preferred_element_type=jnp.float32))


def _combine(x, s0, s1, c0, c1, wr, w0, w1, b):
  blk = pl.BlockSpec((RB, D), lambda i: (i, 0))
  cblk = pl.BlockSpec((RB, CW), lambda i: (i, 0))
  return pl.pallas_call(
      _comb_body,
      grid=(GRID,),
      in_specs=[blk, blk, blk, cblk, cblk,
                _full((D, D)), _full((D, D)), _full((D, D)), _full((1, D))],
      out_specs=pl.BlockSpec((RB, D), lambda i: (i, 0)),
      out_shape=jax.ShapeDtypeStruct((N, D), jnp.float32),
  )(x, s0, s1, c0, c1, wr, w0, w1, b)


def _comb_mlp_body(x_r, s0_r, s1_r, c0_r, c1_r, wr, w0, w1, b,
                   wo1, bo1, wo2, bo2, out_r):
  m0 = s0_r[...] / jnp.maximum(c0_r[...][:, :1], 1.0)
  m1 = s1_r[...] / jnp.maximum(c1_r[...][:, :1], 1.0)
  h = (jnp.dot(x_r[...], wr[...], preferred_element_type=jnp.float32)
       + b[...]
       + jnp.dot(m0, w0[...], preferred_element_type=jnp.float32)
       + jnp.dot(m1, w1[...], preferred_element_type=jnp.float32))
  h = _lk(jnp.dot(h, wo1[...], preferred_element_type=jnp.float32) + bo1[...])
  out_r[...] = (jnp.dot(h, wo2[...], preferred_element_type=jnp.float32)
                + bo2[...])


def _combine_mlp(x, s0, s1, c0, c1, wr, w0, w1, b, wo1, bo1, wo2, bo2):
  blk = pl.BlockSpec((RB, D), lambda i: (i, 0))
  cblk = pl.BlockSpec((RB, CW), lambda i: (i, 0))
  return pl.pallas_call(
      _comb_mlp_body,
      grid=(GRID,),
      in_specs=[blk, blk, blk, cblk, cblk,
                _full((D, D)), _full((D, D)), _full((D, D)), _full((1, D)),
                _full((D, D)), _full((1, D)), _full((D, 2)), _full((1, 2))],
      out_specs=pl.BlockSpec((RB, 2), lambda i: (i, 0)),
      out_shape=jax.ShapeDtypeStruct((N, 2), jnp.float32),
  )(x, s0, s1, c0, c1, wr, w0, w1, b, wo1, bo1, wo2, bo2)


def kernel(des, tweet, num_prop, cat_prop, edge_index, edge_type,
           W_des, b_des, W_tweet, b_tweet, W_num, b_num, W_cat, b_cat,
           W_in, b_in, W_rel, W_root, b_rgcn, W_out1, b_out1, W_out2, b_out2):
  pad = EPAD - E
  src2d = jnp.pad(edge_index[0], (0, pad)).reshape(-1, IDXW)
  dst2d = jnp.pad(edge_index[1], (0, pad)).reshape(-1, IDXW)
  typ2d = jnp.pad(edge_type, (0, pad), constant_values=-1).reshape(-1, IDXW)
  zeros = jnp.zeros((N_TAB, D), jnp.float32)
  zeros_c = jnp.zeros((N_TAB, CW), jnp.float32)
  ones_c = jnp.ones((IDXW, CW), jnp.float32)

  bd = b_des.reshape(1, -1)
  bt = b_tweet.reshape(1, -1)
  bn = b_num.reshape(1, -1)
  bc = b_cat.reshape(1, -1)
  bi = b_in.reshape(1, -1)
  br = b_rgcn.reshape(1, -1)
  bo1 = b_out1.reshape(1, -1)
  bo2 = b_out2.reshape(1, -1)

  x0 = _encoder(des, tweet, num_prop, cat_prop,
                W_des, bd, W_tweet, bt, W_num, bn, W_cat, bc, W_in, bi)

  cnt = _sc_count(dst2d, typ2d, zeros_c, ones_c)[:, :N]
  s_l1 = _sc_segsum(x0, src2d, dst2d, typ2d, zeros)[:, :N]
  x1 = _combine(x0, s_l1[0], s_l1[1], cnt[0], cnt[1],
                W_root, W_rel[0], W_rel[1], br)
  s_l2 = _sc_segsum(x1, src2d, dst2d, typ2d, zeros)[:, :N]
  out = _combine_mlp(x1, s_l2[0], s_l2[1], cnt[0], cnt[1],
                     W_root, W_rel[0], W_rel[1], br,
                     W_out1, bo1, W_out2, bo2)
  return out


# speedup vs baseline: 11.6098x; 1.5891x over previous
import functools

import jax
import jax.numpy as jnp
from jax import lax
from jax.experimental import pallas as pl
from jax.experimental.pallas import tpu as pltpu
from jax.experimental.pallas import tpu_sc as plsc

N = 50000
E = 1600000
D = 32
NUM_REL = 2

NTILES = 16
CHUNK = 512
IDXW = 128
RPC = CHUNK // IDXW
EPAD = -(-E // (NTILES * CHUNK)) * (NTILES * CHUNK)
CH_PER_TILE = EPAD // (NTILES * CHUNK)
N_TAB = 50048
TRASH = 50048
TRASH_N = 4096
TRASH_NC = 8192
RB = 2000
GRID = N // RB


def _lk(x):
  return jnp.where(x > 0, x, 0.01 * x)


_sc_mesh = plsc.VectorSubcoreMesh(core_axis_name="c", subcore_axis_name="s")


@functools.partial(
    pl.kernel,
    out_type=jax.ShapeDtypeStruct((NUM_REL, N_TAB, D), jnp.float32),
    mesh=_sc_mesh,
    compiler_params=pltpu.CompilerParams(use_tc_tiling_on_sc=False),
    scratch_types=[
        pltpu.VMEM((RPC, IDXW), jnp.int32),
        pltpu.VMEM((RPC, IDXW), jnp.int32),
        pltpu.VMEM((RPC, IDXW), jnp.int32),
        pltpu.VMEM((RPC, IDXW), jnp.int32),
        pltpu.VMEM((CHUNK, D), jnp.float32),
        pltpu.VMEM_SHARED((N_TAB + TRASH_N, D), jnp.float32),
        pltpu.SemaphoreType.DMA,
        pltpu.SemaphoreType.DMA,
    ],
)
def _sc_segsum(x_hbm, src_hbm, dst_hbm, typ_hbm, zeros_hbm, out_hbm,
               src_v, dst_v, typ_v, dstp_v, rows_v, table, gsem, ssem):
  c = lax.axis_index("c")
  s = lax.axis_index("s")

  zrows = N_TAB // NTILES
  pltpu.sync_copy(zeros_hbm.at[pl.ds(s * zrows, zrows)],
                  table.at[pl.ds(s * zrows, zrows)])
  plsc.subcore_barrier()

  def body(i, carry):
    row0 = (s * CH_PER_TILE + i) * RPC
    pltpu.sync_copy(src_hbm.at[pl.ds(row0, RPC)], src_v)
    pltpu.sync_copy(dst_hbm.at[pl.ds(row0, RPC)], dst_v)
    pltpu.sync_copy(typ_hbm.at[pl.ds(row0, RPC)], typ_v)
    gcps = [
        pltpu.async_copy(x_hbm.at[src_v.at[j]],
                         rows_v.at[pl.ds(j * IDXW, IDXW)], gsem)
        for j in range(RPC)
    ]
    for j in range(RPC):
      for l in range(IDXW // 16):
        t16 = typ_v[j, pl.ds(l * 16, 16)]
        d16 = dst_v[j, pl.ds(l * 16, 16)]
        dstp_v[j, pl.ds(l * 16, 16)] = jnp.where(
            t16 == c, d16, TRASH + (d16 & (TRASH_N - 1)))
    for cp in gcps:
      cp.wait()
    scps = [
        pltpu.async_copy(rows_v.at[pl.ds(j * IDXW, IDXW)],
                         table.at[dstp_v.at[j]], ssem, add=True)
        for j in range(RPC)
    ]
    for cp in scps:
      cp.wait()
    return carry

  lax.fori_loop(0, CH_PER_TILE, body, 0)
  plsc.subcore_barrier()

  wrows = N_TAB // NTILES
  pltpu.sync_copy(table.at[pl.ds(s * wrows, wrows)],
                  out_hbm.at[c, pl.ds(s * wrows, wrows)])


CW = 8
CCHUNK = 2048
CRPC = CCHUNK // IDXW
CCH_PER_TILE = EPAD // (NTILES * CCHUNK)


@functools.partial(
    pl.kernel,
    out_type=jax.ShapeDtypeStruct((NUM_REL, N_TAB, CW), jnp.float32),
    mesh=_sc_mesh,
    compiler_params=pltpu.CompilerParams(use_tc_tiling_on_sc=False),
    scratch_types=[
        pltpu.VMEM((CRPC, IDXW), jnp.int32),
        pltpu.VMEM((CRPC, IDXW), jnp.int32),
        pltpu.VMEM((CRPC, IDXW), jnp.int32),
        pltpu.VMEM((IDXW, CW), jnp.float32),
        pltpu.VMEM_SHARED((N_TAB + TRASH_NC, CW), jnp.float32),
        pltpu.SemaphoreType.DMA,
    ],
)
def _sc_count(dst_hbm, typ_hbm, zeros_hbm, ones_hbm, out_hbm,
              dst_v, typ_v, dstp_v, ones_v, table, ssem):
  c = lax.axis_index("c")
  s = lax.axis_index("s")

  zrows = N_TAB // NTILES
  pltpu.sync_copy(zeros_hbm.at[pl.ds(s * zrows, zrows)],
                  table.at[pl.ds(s * zrows, zrows)])
  pltpu.sync_copy(ones_hbm, ones_v)
  plsc.subcore_barrier()

  def body(i, carry):
    row0 = (s * CCH_PER_TILE + i) * CRPC
    pltpu.sync_copy(dst_hbm.at[pl.ds(row0, CRPC)], dst_v)
    pltpu.sync_copy(typ_hbm.at[pl.ds(row0, CRPC)], typ_v)
    for j in range(CRPC):
      for l in range(IDXW // 16):
        t16 = typ_v[j, pl.ds(l * 16, 16)]
        d16 = dst_v[j, pl.ds(l * 16, 16)]
        dstp_v[j, pl.ds(l * 16, 16)] = jnp.where(
            t16 == c, d16, TRASH + (d16 & (TRASH_NC - 1)))
    scps = [
        pltpu.async_copy(ones_v, table.at[dstp_v.at[j]], ssem, add=True)
        for j in range(CRPC)
    ]
    for cp in scps:
      cp.wait()
    return carry

  lax.fori_loop(0, CCH_PER_TILE, body, 0)
  plsc.subcore_barrier()

  wrows = N_TAB // NTILES
  pltpu.sync_copy(table.at[pl.ds(s * wrows, wrows)],
                  out_hbm.at[c, pl.ds(s * wrows, wrows)])


def _enc_body(des_r, tw_r, np_r, cp_r, wd, bd, wt, bt, wn, bn, wc, bc,
              win, bin_, out_r):
  d = _lk(jnp.dot(des_r[...], wd[...], preferred_element_type=jnp.float32)
          + bd[...])
  t = _lk(jnp.dot(tw_r[...], wt[...], preferred_element_type=jnp.float32)
          + bt[...])
  n = _lk(jnp.dot(np_r[...], wn[...], preferred_element_type=jnp.float32)
          + bn[...])
  cc = _lk(jnp.dot(cp_r[...], wc[...], preferred_element_type=jnp.float32)
           + bc[...])
  x = jnp.concatenate([d, t, n, cc], axis=1)
  out_r[...] = _lk(jnp.dot(x, win[...], preferred_element_type=jnp.float32)
                   + bin_[...])


def _full(shape):
  return pl.BlockSpec(shape, lambda i: (0, 0))


def _encoder(des, tweet, num_prop, cat_prop, wd, bd, wt, bt, wn, bn, wc, bc,
             win, bin_):
  return pl.pallas_call(
      _enc_body,
      grid=(GRID,),
      in_specs=[
          pl.BlockSpec((RB, 768), lambda i: (i, 0)),
          pl.BlockSpec((RB, 768), lambda i: (i, 0)),
          pl.BlockSpec((RB, 6), lambda i: (i, 0)),
          pl.BlockSpec((RB, 3), lambda i: (i, 0)),
          _full((768, D // 4)), _full((1, D // 4)),
          _full((768, D // 4)), _full((1, D // 4)),
          _full((6, D // 4)), _full((1, D // 4)),
          _full((3, D // 4)), _full((1, D // 4)),
          _full((D, D)), _full((1, D)),
      ],
      out_specs=pl.BlockSpec((RB, D), lambda i: (i, 0)),
      out_shape=jax.ShapeDtypeStruct((N, D), jnp.float32),
  )(des, tweet, num_prop, cat_prop, wd, bd, wt, bt, wn, bn, wc, bc, win, bin_)


def _comb_body(x_r, s0_r, s1_r, c0_r, c1_r, wr, w0, w1, b, out_r):
  m0 = s0_r[...] / jnp.maximum(c0_r[...][:, :1], 1.0)
  m1 = s1_r[...] / jnp.maximum(c1_r[...][:, :1], 1.0)
  out_r[...] = (jnp.dot(x_r[...], wr[...], preferred_element_type=jnp.float32)
                + b[...]
                + jnp.dot(m0, w0[...], preferred_element_type=jnp.float32)
                + jnp.dot(m1, w1[...], preferred_element_type=jnp.float32))


def _combine(x, s0, s1, c0, c1, wr, w0, w1, b):
  blk = pl.BlockSpec((RB, D), lambda i: (i, 0))
  cblk = pl.BlockSpec((RB, CW), lambda i: (i, 0))
  return pl.pallas_call(
      _comb_body,
      grid=(GRID,),
      in_specs=[blk, blk, blk, cblk, cblk,
                _full((D, D)), _full((D, D)), _full((D, D)), _full((1, D))],
      out_specs=pl.BlockSpec((RB, D), lambda i: (i, 0)),
      out_shape=jax.ShapeDtypeStruct((N, D), jnp.float32),
  )(x, s0, s1, c0, c1, wr, w0, w1, b)


def _comb_mlp_body(x_r, s0_r, s1_r, c0_r, c1_r, wr, w0, w1, b,
                   wo1, bo1, wo2, bo2, out_r):
  m0 = s0_r[...] / jnp.maximum(c0_r[...][:, :1], 1.0)
  m1 = s1_r[...] / jnp.maximum(c1_r[...][:, :1], 1.0)
  h = (jnp.dot(x_r[...], wr[...], preferred_element_type=jnp.float32)
       + b[...]
       + jnp.dot(m0, w0[...], preferred_element_type=jnp.float32)
       + jnp.dot(m1, w1[...], preferred_element_type=jnp.float32))
  h = _lk(jnp.dot(h, wo1[...], preferred_element_type=jnp.float32) + bo1[...])
  out_r[...] = (jnp.dot(h, wo2[...], preferred_element_type=jnp.float32)
                + bo2[...])


def _combine_mlp(x, s0, s1, c0, c1, wr, w0, w1, b, wo1, bo1, wo2, bo2):
  blk = pl.BlockSpec((RB, D), lambda i: (i, 0))
  cblk = pl.BlockSpec((RB, CW), lambda i: (i, 0))
  return pl.pallas_call(
      _comb_mlp_body,
      grid=(GRID,),
      in_specs=[blk, blk, blk, cblk, cblk,
                _full((D, D)), _full((D, D)), _full((D, D)), _full((1, D)),
                _full((D, D)), _full((1, D)), _full((D, 2)), _full((1, 2))],
      out_specs=pl.BlockSpec((RB, 2), lambda i: (i, 0)),
      out_shape=jax.ShapeDtypeStruct((N, 2), jnp.float32),
  )(x, s0, s1, c0, c1, wr, w0, w1, b, wo1, bo1, wo2, bo2)


def kernel(des, tweet, num_prop, cat_prop, edge_index, edge_type,
           W_des, b_des, W_tweet, b_tweet, W_num, b_num, W_cat, b_cat,
           W_in, b_in, W_rel, W_root, b_rgcn, W_out1, b_out1, W_out2, b_out2):
  pad = EPAD - E
  src2d = jnp.pad(edge_index[0], (0, pad)).reshape(-1, IDXW)
  dst2d = jnp.pad(edge_index[1], (0, pad)).reshape(-1, IDXW)
  typ2d = jnp.pad(edge_type, (0, pad), constant_values=-1).reshape(-1, IDXW)
  zeros = jnp.zeros((N_TAB, D), jnp.float32)
  zeros_c = jnp.zeros((N_TAB, CW), jnp.float32)
  ones_c = jnp.ones((IDXW, CW), jnp.float32)

  bd = b_des.reshape(1, -1)
  bt = b_tweet.reshape(1, -1)
  bn = b_num.reshape(1, -1)
  bc = b_cat.reshape(1, -1)
  bi = b_in.reshape(1, -1)
  br = b_rgcn.reshape(1, -1)
  bo1 = b_out1.reshape(1, -1)
  bo2 = b_out2.reshape(1, -1)

  x0 = _encoder(des, tweet, num_prop, cat_prop,
                W_des, bd, W_tweet, bt, W_num, bn, W_cat, bc, W_in, bi)

  cnt = _sc_count(dst2d, typ2d, zeros_c, ones_c)[:, :N]
  s_l1 = _sc_segsum(x0, src2d, dst2d, typ2d, zeros)[:, :N]
  x1 = _combine(x0, s_l1[0], s_l1[1], cnt[0], cnt[1],
                W_root, W_rel[0], W_rel[1], br)
  s_l2 = _sc_segsum(x1, src2d, dst2d, typ2d, zeros)[:, :N]
  out = _combine_mlp(x1, s_l2[0], s_l2[1], cnt[0], cnt[1],
                     W_root, W_rel[0], W_rel[1], br,
                     W_out1, bo1, W_out2, bo2)
  return out


# speedup vs baseline: 12.0980x; 1.0421x over previous
import functools

import jax
import jax.numpy as jnp
from jax import lax
from jax.experimental import pallas as pl
from jax.experimental.pallas import tpu as pltpu
from jax.experimental.pallas import tpu_sc as plsc

N = 50000
E = 1600000
D = 32
NUM_REL = 2

NTILES = 16
CHUNK = 512
IDXW = 128
RPC = CHUNK // IDXW
SROWS = E // IDXW
NCH = E // CHUNK
CH_FULL = NCH // NTILES
CH_EXTRA = NCH % NTILES
N_TAB = 50048
TRASH = 50048
TRASH_N = 4096
TRASH_NC = 8192
RB = 2000
GRID = N // RB


def _lk(x):
  return jnp.where(x > 0, x, 0.01 * x)


_sc_mesh = plsc.VectorSubcoreMesh(core_axis_name="c", subcore_axis_name="s")


@functools.partial(
    pl.kernel,
    out_type=jax.ShapeDtypeStruct((NUM_REL, N_TAB, D), jnp.float32),
    mesh=_sc_mesh,
    compiler_params=pltpu.CompilerParams(use_tc_tiling_on_sc=False),
    scratch_types=[
        pltpu.VMEM((RPC, IDXW), jnp.int32),
        pltpu.VMEM((RPC, IDXW), jnp.int32),
        pltpu.VMEM((RPC, IDXW), jnp.int32),
        pltpu.VMEM((RPC, IDXW), jnp.int32),
        pltpu.VMEM((CHUNK, D), jnp.float32),
        pltpu.VMEM_SHARED((N_TAB + TRASH_N, D), jnp.float32),
        pltpu.SemaphoreType.DMA,
        pltpu.SemaphoreType.DMA,
    ],
)
def _sc_segsum(x_hbm, edge_hbm, typ_hbm, zeros_hbm, out_hbm,
               src_v, dst_v, typ_v, dstp_v, rows_v, table, gsem, ssem):
  c = lax.axis_index("c")
  s = lax.axis_index("s")

  zrows = N_TAB // NTILES
  pltpu.sync_copy(zeros_hbm, table.at[pl.ds(s * zrows, zrows)])
  plsc.subcore_barrier()

  def chunk(g):
    row0 = g * RPC
    pltpu.sync_copy(edge_hbm.at[pl.ds(row0, RPC)], src_v)
    pltpu.sync_copy(edge_hbm.at[pl.ds(SROWS + row0, RPC)], dst_v)
    pltpu.sync_copy(typ_hbm.at[pl.ds(row0, RPC)], typ_v)
    gcps = [
        pltpu.async_copy(x_hbm.at[src_v.at[j]],
                         rows_v.at[pl.ds(j * IDXW, IDXW)], gsem)
        for j in range(RPC)
    ]
    for j in range(RPC):
      for l in range(IDXW // 16):
        t16 = typ_v[j, pl.ds(l * 16, 16)]
        d16 = dst_v[j, pl.ds(l * 16, 16)]
        dstp_v[j, pl.ds(l * 16, 16)] = jnp.where(
            t16 == c, d16, TRASH + (d16 & (TRASH_N - 1)))
    for cp in gcps:
      cp.wait()
    scps = [
        pltpu.async_copy(rows_v.at[pl.ds(j * IDXW, IDXW)],
                         table.at[dstp_v.at[j]], ssem, add=True)
        for j in range(RPC)
    ]
    for cp in scps:
      cp.wait()

  def body(i, carry):
    chunk(i * NTILES + s)
    return carry

  lax.fori_loop(0, CH_FULL, body, 0)
  @pl.when(s < CH_EXTRA)
  def _tail():
    chunk(CH_FULL * NTILES + s)
  plsc.subcore_barrier()

  wrows = N_TAB // NTILES
  pltpu.sync_copy(table.at[pl.ds(s * wrows, wrows)],
                  out_hbm.at[c, pl.ds(s * wrows, wrows)])


CW = 8


@functools.partial(
    pl.kernel,
    out_type=jax.ShapeDtypeStruct((NUM_REL, N_TAB, CW), jnp.float32),
    mesh=_sc_mesh,
    compiler_params=pltpu.CompilerParams(use_tc_tiling_on_sc=False),
    scratch_types=[
        pltpu.VMEM((RPC, IDXW), jnp.int32),
        pltpu.VMEM((RPC, IDXW), jnp.int32),
        pltpu.VMEM((RPC, IDXW), jnp.int32),
        pltpu.VMEM((IDXW, CW), jnp.float32),
        pltpu.VMEM_SHARED((N_TAB + TRASH_NC, CW), jnp.float32),
        pltpu.SemaphoreType.DMA,
    ],
)
def _sc_count(edge_hbm, typ_hbm, zeros_hbm, ones_hbm, out_hbm,
              dst_v, typ_v, dstp_v, ones_v, table, ssem):
  c = lax.axis_index("c")
  s = lax.axis_index("s")

  zrows = N_TAB // NTILES
  pltpu.sync_copy(zeros_hbm, table.at[pl.ds(s * zrows, zrows)])
  pltpu.sync_copy(ones_hbm, ones_v)
  plsc.subcore_barrier()

  def chunk(g):
    row0 = g * RPC
    pltpu.sync_copy(edge_hbm.at[pl.ds(SROWS + row0, RPC)], dst_v)
    pltpu.sync_copy(typ_hbm.at[pl.ds(row0, RPC)], typ_v)
    for j in range(RPC):
      for l in range(IDXW // 16):
        t16 = typ_v[j, pl.ds(l * 16, 16)]
        d16 = dst_v[j, pl.ds(l * 16, 16)]
        dstp_v[j, pl.ds(l * 16, 16)] = jnp.where(
            t16 == c, d16, TRASH + (d16 & (TRASH_NC - 1)))
    scps = [
        pltpu.async_copy(ones_v, table.at[dstp_v.at[j]], ssem, add=True)
        for j in range(RPC)
    ]
    for cp in scps:
      cp.wait()

  def body(i, carry):
    chunk(i * NTILES + s)
    return carry

  lax.fori_loop(0, CH_FULL, body, 0)
  @pl.when(s < CH_EXTRA)
  def _tail():
    chunk(CH_FULL * NTILES + s)
  plsc.subcore_barrier()

  wrows = N_TAB // NTILES
  pltpu.sync_copy(table.at[pl.ds(s * wrows, wrows)],
                  out_hbm.at[c, pl.ds(s * wrows, wrows)])


def _enc_body(des_r, tw_r, np_r, cp_r, wd, bd, wt, bt, wn, bn, wc, bc,
              win, bin_, out_r):
  d = _lk(jnp.dot(des_r[...], wd[...], preferred_element_type=jnp.float32)
          + bd[...])
  t = _lk(jnp.dot(tw_r[...], wt[...], preferred_element_type=jnp.float32)
          + bt[...])
  n = _lk(jnp.dot(np_r[...], wn[...], preferred_element_type=jnp.float32)
          + bn[...])
  cc = _lk(jnp.dot(cp_r[...], wc[...], preferred_element_type=jnp.float32)
           + bc[...])
  x = jnp.concatenate([d, t, n, cc], axis=1)
  out_r[...] = _lk(jnp.dot(x, win[...], preferred_element_type=jnp.float32)
                   + bin_[...])


def _full(shape):
  return pl.BlockSpec(shape, lambda i: (0, 0))


def _encoder(des, tweet, num_prop, cat_prop, wd, bd, wt, bt, wn, bn, wc, bc,
             win, bin_):
  return pl.pallas_call(
      _enc_body,
      grid=(GRID,),
      in_specs=[
          pl.BlockSpec((RB, 768), lambda i: (i, 0)),
          pl.BlockSpec((RB, 768), lambda i: (i, 0)),
          pl.BlockSpec((RB, 6), lambda i: (i, 0)),
          pl.BlockSpec((RB, 3), lambda i: (i, 0)),
          _full((768, D // 4)), _full((1, D // 4)),
          _full((768, D // 4)), _full((1, D // 4)),
          _full((6, D // 4)), _full((1, D // 4)),
          _full((3, D // 4)), _full((1, D // 4)),
          _full((D, D)), _full((1, D)),
      ],
      out_specs=pl.BlockSpec((RB, D), lambda i: (i, 0)),
      out_shape=jax.ShapeDtypeStruct((N, D), jnp.float32),
  )(des, tweet, num_prop, cat_prop, wd, bd, wt, bt, wn, bn, wc, bc, win, bin_)


def _mean_terms(s_r, c_r):
  m0 = s_r[0] / jnp.maximum(c_r[0][:, :1], 1.0)
  m1 = s_r[1] / jnp.maximum(c_r[1][:, :1], 1.0)
  return m0, m1


_sblk = pl.BlockSpec((NUM_REL, RB, D), lambda i: (0, i, 0))
_cblk = pl.BlockSpec((NUM_REL, RB, CW), lambda i: (0, i, 0))


def _comb_body(x_r, s_r, c_r, wr, w0, w1, b, out_r):
  m0, m1 = _mean_terms(s_r, c_r)
  out_r[...] = (jnp.dot(x_r[...], wr[...], preferred_element_type=jnp.float32)
                + b[...]
                + jnp.dot(m0, w0[...], preferred_element_type=jnp.float32)
                + jnp.dot(m1, w1[...], preferred_element_type=jnp.float32))


def _combine(x, s_, c_, wr, w0, w1, b):
  blk = pl.BlockSpec((RB, D), lambda i: (i, 0))
  return pl.pallas_call(
      _comb_body,
      grid=(GRID,),
      in_specs=[blk, _sblk, _cblk,
                _full((D, D)), _full((D, D)), _full((D, D)), _full((1, D))],
      out_specs=pl.BlockSpec((RB, D), lambda i: (i, 0)),
      out_shape=jax.ShapeDtypeStruct((N, D), jnp.float32),
  )(x, s_, c_, wr, w0, w1, b)


def _comb_mlp_body(x_r, s_r, c_r, wr, w0, w1, b, wo1, bo1, wo2, bo2, out_r):
  m0, m1 = _mean_terms(s_r, c_r)
  h = (jnp.dot(x_r[...], wr[...], preferred_element_type=jnp.float32)
       + b[...]
       + jnp.dot(m0, w0[...], preferred_element_type=jnp.float32)
       + jnp.dot(m1, w1[...], preferred_element_type=jnp.float32))
  h = _lk(jnp.dot(h, wo1[...], preferred_element_type=jnp.float32) + bo1[...])
  out_r[...] = (jnp.dot(h, wo2[...], preferred_element_type=jnp.float32)
                + bo2[...])


def _combine_mlp(x, s_, c_, wr, w0, w1, b, wo1, bo1, wo2, bo2):
  blk = pl.BlockSpec((RB, D), lambda i: (i, 0))
  return pl.pallas_call(
      _comb_mlp_body,
      grid=(GRID,),
      in_specs=[blk, _sblk, _cblk,
                _full((D, D)), _full((D, D)), _full((D, D)), _full((1, D)),
                _full((D, D)), _full((1, D)), _full((D, 2)), _full((1, 2))],
      out_specs=pl.BlockSpec((RB, 2), lambda i: (i, 0)),
      out_shape=jax.ShapeDtypeStruct((N, 2), jnp.float32),
  )(x, s_, c_, wr, w0, w1, b, wo1, bo1, wo2, bo2)


def kernel(des, tweet, num_prop, cat_prop, edge_index, edge_type,
           W_des, b_des, W_tweet, b_tweet, W_num, b_num, W_cat, b_cat,
           W_in, b_in, W_rel, W_root, b_rgcn, W_out1, b_out1, W_out2, b_out2):
  edge2d = edge_index.reshape(2 * SROWS, IDXW)
  typ2d = edge_type.reshape(SROWS, IDXW)
  zeros = jnp.zeros((N_TAB // NTILES, D), jnp.float32)
  zeros_c = jnp.zeros((N_TAB // NTILES, CW), jnp.float32)
  ones_c = jnp.ones((IDXW, CW), jnp.float32)

  bd = b_des.reshape(1, -1)
  bt = b_tweet.reshape(1, -1)
  bn = b_num.reshape(1, -1)
  bc = b_cat.reshape(1, -1)
  bi = b_in.reshape(1, -1)
  br = b_rgcn.reshape(1, -1)
  bo1 = b_out1.reshape(1, -1)
  bo2 = b_out2.reshape(1, -1)

  x0 = _encoder(des, tweet, num_prop, cat_prop,
                W_des, bd, W_tweet, bt, W_num, bn, W_cat, bc, W_in, bi)

  cnt = _sc_count(edge2d, typ2d, zeros_c, ones_c)
  s_l1 = _sc_segsum(x0, edge2d, typ2d, zeros)
  x1 = _combine(x0, s_l1, cnt, W_root, W_rel[0], W_rel[1], br)
  s_l2 = _sc_segsum(x1, edge2d, typ2d, zeros)
  out = _combine_mlp(x1, s_l2, cnt, W_root, W_rel[0], W_rel[1], br,
                     W_out1, bo1, W_out2, bo2)
  return out


# speedup vs baseline: 16.5524x; 1.3682x over previous
import functools

import jax
import jax.numpy as jnp
from jax import lax
from jax.experimental import pallas as pl
from jax.experimental.pallas import tpu as pltpu
from jax.experimental.pallas import tpu_sc as plsc

N = 50000
E = 1600000
D = 32
NUM_REL = 2

NTILES = 16
CHUNK = 512
IDXW = 128
RPC = CHUNK // IDXW
SROWS = E // IDXW
NCH = E // CHUNK
CH_FULL = NCH // NTILES
CH_EXTRA = NCH % NTILES
CAP_CH = CH_FULL + 1
REG_ROWS = CAP_CH * RPC
CROWS = NTILES * REG_ROWS
BUF = 3 * CHUNK
N_TAB = 50048
TRASH = 50048
TRASH_N = 4096
TRASH_NC = 8192
CW = 8
RB = 2000
GRID = N // RB


def _lk(x):
  return jnp.where(x > 0, x, 0.01 * x)


_sc_mesh = plsc.VectorSubcoreMesh(core_axis_name="c", subcore_axis_name="s")
_sc_params = pltpu.CompilerParams(use_tc_tiling_on_sc=False)
_sc_params_nl = pltpu.CompilerParams(
    use_tc_tiling_on_sc=False, needs_layout_passes=False)


@functools.partial(
    pl.kernel,
    out_type=[
        jax.ShapeDtypeStruct((NUM_REL, CROWS, IDXW), jnp.int32),
        jax.ShapeDtypeStruct((NUM_REL, CROWS, IDXW), jnp.int32),
        jax.ShapeDtypeStruct((NUM_REL, NTILES, 16), jnp.int32),
        jax.ShapeDtypeStruct((NUM_REL, N_TAB, CW), jnp.float32),
    ],
    mesh=_sc_mesh,
    compiler_params=_sc_params_nl,
    scratch_types=[
        pltpu.VMEM((RPC, IDXW), jnp.int32),
        pltpu.VMEM((RPC, IDXW), jnp.int32),
        pltpu.VMEM((RPC, IDXW), jnp.int32),
        pltpu.VMEM((RPC, IDXW), jnp.int32),
        pltpu.VMEM((IDXW, CW), jnp.float32),
        pltpu.VMEM((BUF,), jnp.int32),
        pltpu.VMEM((BUF,), jnp.int32),
        pltpu.VMEM((16,), jnp.int32),
        pltpu.VMEM_SHARED((N_TAB + TRASH_NC, CW), jnp.float32),
        pltpu.SemaphoreType.DMA,
    ],
)
def _sc_partition(edge_hbm, typ_hbm, zeros_hbm, ones_hbm,
                  srcc_hbm, dstc_hbm, nch_hbm, cnt_hbm,
                  src_v, dst_v, typ_v, dstp_v, ones_v, bsrc, bdst, cbuf,
                  table, ssem):
  c = lax.axis_index("c")
  s = lax.axis_index("s")

  zrows = N_TAB // NTILES
  pltpu.sync_copy(zeros_hbm, table.at[pl.ds(s * zrows, zrows)])
  pltpu.sync_copy(ones_hbm, ones_v)
  plsc.subcore_barrier()

  base_row = s * REG_ROWS
  iota16 = jnp.arange(16, dtype=jnp.int32)

  def chunk(g, off, n):
    row0 = g * RPC
    pltpu.sync_copy(edge_hbm.at[pl.ds(row0, RPC)], src_v)
    pltpu.sync_copy(edge_hbm.at[pl.ds(SROWS + row0, RPC)], dst_v)
    pltpu.sync_copy(typ_hbm.at[pl.ds(row0, RPC)], typ_v)
    for j in range(RPC):
      for l in range(IDXW // 16):
        t16 = typ_v[j, pl.ds(l * 16, 16)]
        d16 = dst_v[j, pl.ds(l * 16, 16)]
        dstp_v[j, pl.ds(l * 16, 16)] = jnp.where(
            t16 == c, d16, TRASH + (d16 & (TRASH_NC - 1)))
    scps = [
        pltpu.async_copy(ones_v, table.at[dstp_v.at[j]], ssem, add=True)
        for j in range(RPC)
    ]
    for j in range(RPC):
      for l in range(IDXW // 16):
        t16 = typ_v[j, pl.ds(l * 16, 16)]
        m = t16 == c
        s16 = src_v[j, pl.ds(l * 16, 16)]
        d16 = dst_v[j, pl.ds(l * 16, 16)]
        pos = off + plsc.cumsum(m.astype(jnp.int32)) - 1
        plsc.store_scatter(bsrc, [pos], s16, mask=m)
        plsc.store_scatter(bdst, [pos], d16, mask=m)
        off = off + plsc.all_reduce_population_count(m)
    for cp in scps:
      cp.wait()
    off_sc = off[0]
    do_flush = off_sc >= CHUNK

    @pl.when(do_flush)
    def _flush():
      for k in range(RPC):
        pltpu.sync_copy(bsrc.at[pl.ds(k * IDXW, IDXW)],
                        srcc_hbm.at[c, base_row + n * RPC + k])
        pltpu.sync_copy(bdst.at[pl.ds(k * IDXW, IDXW)],
                        dstc_hbm.at[c, base_row + n * RPC + k])
      for k in range(CHUNK // 16):
        bsrc[pl.ds(k * 16, 16)] = bsrc[pl.ds(CHUNK + k * 16, 16)]
        bdst[pl.ds(k * 16, 16)] = bdst[pl.ds(CHUNK + k * 16, 16)]

    off = jnp.where(off >= CHUNK, off - CHUNK, off)
    n = n + do_flush.astype(jnp.int32)
    return off, n

  def body(i, carry):
    return chunk(i * NTILES + s, *carry)

  off, n = lax.fori_loop(
      0, CH_FULL, body, (jnp.zeros((16,), jnp.int32), jnp.int32(0)))

  def tail(carry):
    return chunk(CH_FULL * NTILES + s, *carry)

  off, n = lax.cond(s < CH_EXTRA, tail, lambda carry: carry, (off, n))

  off_sc = off[0]

  @pl.when(off_sc > 0)
  def _last():
    for k in range(CHUNK // 16):
      bsrc[pl.ds(off_sc + k * 16, 16)] = jnp.zeros((16,), jnp.int32)
      bdst[pl.ds(off_sc + k * 16, 16)] = TRASH + iota16 + k * 16
    for k in range(RPC):
      pltpu.sync_copy(bsrc.at[pl.ds(k * IDXW, IDXW)],
                      srcc_hbm.at[c, base_row + n * RPC + k])
      pltpu.sync_copy(bdst.at[pl.ds(k * IDXW, IDXW)],
                      dstc_hbm.at[c, base_row + n * RPC + k])

  n_final = n + (off_sc > 0).astype(jnp.int32)
  cbuf[...] = jnp.broadcast_to(n_final, (16,)).astype(jnp.int32)
  pltpu.sync_copy(cbuf, nch_hbm.at[c, s])
  plsc.subcore_barrier()

  wrows = N_TAB // NTILES
  pltpu.sync_copy(table.at[pl.ds(s * wrows, wrows)],
                  cnt_hbm.at[c, pl.ds(s * wrows, wrows)])


@functools.partial(
    pl.kernel,
    out_type=jax.ShapeDtypeStruct((NUM_REL, N_TAB, D), jnp.float32),
    mesh=_sc_mesh,
    compiler_params=_sc_params,
    scratch_types=[
        pltpu.VMEM((RPC, IDXW), jnp.int32),
        pltpu.VMEM((RPC, IDXW), jnp.int32),
        pltpu.VMEM((CHUNK, D), jnp.float32),
        pltpu.VMEM((16,), jnp.int32),
        pltpu.VMEM_SHARED((N_TAB + TRASH_N, D), jnp.float32),
        pltpu.SemaphoreType.DMA,
        pltpu.SemaphoreType.DMA,
    ],
)
def _sc_segsum(x_hbm, srcc_hbm, dstc_hbm, nch_hbm, zeros_hbm, out_hbm,
               src_v, dst_v, rows_v, cbuf, table, gsem, ssem):
  c = lax.axis_index("c")
  s = lax.axis_index("s")

  zrows = N_TAB // NTILES
  pltpu.sync_copy(zeros_hbm, table.at[pl.ds(s * zrows, zrows)])
  pltpu.sync_copy(nch_hbm.at[c, s], cbuf)
  plsc.subcore_barrier()

  nchunks = cbuf[...][0]
  base_row = s * REG_ROWS

  def body(i, carry):
    row0 = base_row + i * RPC
    pltpu.sync_copy(srcc_hbm.at[c, pl.ds(row0, RPC)], src_v)
    pltpu.sync_copy(dstc_hbm.at[c, pl.ds(row0, RPC)], dst_v)
    gcps = [
        pltpu.async_copy(x_hbm.at[src_v.at[j]],
                         rows_v.at[pl.ds(j * IDXW, IDXW)], gsem)
        for j in range(RPC)
    ]
    for cp in gcps:
      cp.wait()
    scps = [
        pltpu.async_copy(rows_v.at[pl.ds(j * IDXW, IDXW)],
                         table.at[dst_v.at[j]], ssem, add=True)
        for j in range(RPC)
    ]
    for cp in scps:
      cp.wait()
    return carry

  lax.fori_loop(0, nchunks, body, 0)
  plsc.subcore_barrier()

  wrows = N_TAB // NTILES
  pltpu.sync_copy(table.at[pl.ds(s * wrows, wrows)],
                  out_hbm.at[c, pl.ds(s * wrows, wrows)])


def _enc_body(des_r, tw_r, np_r, cp_r, wd, bd, wt, bt, wn, bn, wc, bc,
              win, bin_, out_r):
  d = _lk(jnp.dot(des_r[...], wd[...], preferred_element_type=jnp.float32)
          + bd[...])
  t = _lk(jnp.dot(tw_r[...], wt[...], preferred_element_type=jnp.float32)
          + bt[...])
  n = _lk(jnp.dot(np_r[...], wn[...], preferred_element_type=jnp.float32)
          + bn[...])
  cc = _lk(jnp.dot(cp_r[...], wc[...], preferred_element_type=jnp.float32)
           + bc[...])
  x = jnp.concatenate([d, t, n, cc], axis=1)
  out_r[...] = _lk(jnp.dot(x, win[...], preferred_element_type=jnp.float32)
                   + bin_[...])


def _full(shape):
  return pl.BlockSpec(shape, lambda i: (0, 0))


def _encoder(des, tweet, num_prop, cat_prop, wd, bd, wt, bt, wn, bn, wc, bc,
             win, bin_):
  return pl.pallas_call(
      _enc_body,
      grid=(GRID,),
      in_specs=[
          pl.BlockSpec((RB, 768), lambda i: (i, 0)),
          pl.BlockSpec((RB, 768), lambda i: (i, 0)),
          pl.BlockSpec((RB, 6), lambda i: (i, 0)),
          pl.BlockSpec((RB, 3), lambda i: (i, 0)),
          _full((768, D // 4)), _full((1, D // 4)),
          _full((768, D // 4)), _full((1, D // 4)),
          _full((6, D // 4)), _full((1, D // 4)),
          _full((3, D // 4)), _full((1, D // 4)),
          _full((D, D)), _full((1, D)),
      ],
      out_specs=pl.BlockSpec((RB, D), lambda i: (i, 0)),
      out_shape=jax.ShapeDtypeStruct((N, D), jnp.float32),
  )(des, tweet, num_prop, cat_prop, wd, bd, wt, bt, wn, bn, wc, bc, win, bin_)


def _mean_terms(s_r, c_r):
  m0 = s_r[0] / jnp.maximum(c_r[0][:, :1], 1.0)
  m1 = s_r[1] / jnp.maximum(c_r[1][:, :1], 1.0)
  return m0, m1


_sblk = pl.BlockSpec((NUM_REL, RB, D), lambda i: (0, i, 0))
_cblk = pl.BlockSpec((NUM_REL, RB, CW), lambda i: (0, i, 0))


def _comb_body(x_r, s_r, c_r, wr, w0, w1, b, out_r):
  m0, m1 = _mean_terms(s_r, c_r)
  out_r[...] = (jnp.dot(x_r[...], wr[...], preferred_element_type=jnp.float32)
                + b[...]
                + jnp.dot(m0, w0[...], preferred_element_type=jnp.float32)
                + jnp.dot(m1, w1[...], preferred_element_type=jnp.float32))


def _combine(x, s_, c_, wr, w0, w1, b):
  blk = pl.BlockSpec((RB, D), lambda i: (i, 0))
  return pl.pallas_call(
      _comb_body,
      grid=(GRID,),
      in_specs=[blk, _sblk, _cblk,
                _full((D, D)), _full((D, D)), _full((D, D)), _full((1, D))],
      out_specs=pl.BlockSpec((RB, D), lambda i: (i, 0)),
      out_shape=jax.ShapeDtypeStruct((N, D), jnp.float32),
  )(x, s_, c_, wr, w0, w1, b)


def _comb_mlp_body(x_r, s_r, c_r, wr, w0, w1, b, wo1, bo1, wo2, bo2, out_r):
  m0, m1 = _mean_terms(s_r, c_r)
  h = (jnp.dot(x_r[...], wr[...], preferred_element_type=jnp.float32)
       + b[...]
       + jnp.dot(m0, w0[...], preferred_element_type=jnp.float32)
       + jnp.dot(m1, w1[...], preferred_element_type=jnp.float32))
  h = _lk(jnp.dot(h, wo1[...], preferred_element_type=jnp.float32) + bo1[...])
  out_r[...] = (jnp.dot(h, wo2[...], preferred_element_type=jnp.float32)
                + bo2[...])


def _combine_mlp(x, s_, c_, wr, w0, w1, b, wo1, bo1, wo2, bo2):
  blk = pl.BlockSpec((RB, D), lambda i: (i, 0))
  return pl.pallas_call(
      _comb_mlp_body,
      grid=(GRID,),
      in_specs=[blk, _sblk, _cblk,
                _full((D, D)), _full((D, D)), _full((D, D)), _full((1, D)),
                _full((D, D)), _full((1, D)), _full((D, 2)), _full((1, 2))],
      out_specs=pl.BlockSpec((RB, 2), lambda i: (i, 0)),
      out_shape=jax.ShapeDtypeStruct((N, 2), jnp.float32),
  )(x, s_, c_, wr, w0, w1, b, wo1, bo1, wo2, bo2)


def kernel(des, tweet, num_prop, cat_prop, edge_index, edge_type,
           W_des, b_des, W_tweet, b_tweet, W_num, b_num, W_cat, b_cat,
           W_in, b_in, W_rel, W_root, b_rgcn, W_out1, b_out1, W_out2, b_out2):
  edge2d = edge_index.reshape(2 * SROWS, IDXW)
  typ2d = edge_type.reshape(SROWS, IDXW)
  zeros = jnp.zeros((N_TAB // NTILES, D), jnp.float32)
  zeros_c = jnp.zeros((N_TAB // NTILES, CW), jnp.float32)
  ones_c = jnp.ones((IDXW, CW), jnp.float32)

  bd = b_des.reshape(1, -1)
  bt = b_tweet.reshape(1, -1)
  bn = b_num.reshape(1, -1)
  bc = b_cat.reshape(1, -1)
  bi = b_in.reshape(1, -1)
  br = b_rgcn.reshape(1, -1)
  bo1 = b_out1.reshape(1, -1)
  bo2 = b_out2.reshape(1, -1)

  x0 = _encoder(des, tweet, num_prop, cat_prop,
                W_des, bd, W_tweet, bt, W_num, bn, W_cat, bc, W_in, bi)

  srcc, dstc, nch, cnt = _sc_partition(edge2d, typ2d, zeros_c, ones_c)
  s_l1 = _sc_segsum(x0, srcc, dstc, nch, zeros)
  x1 = _combine(x0, s_l1, cnt, W_root, W_rel[0], W_rel[1], br)
  s_l2 = _sc_segsum(x1, srcc, dstc, nch, zeros)
  out = _combine_mlp(x1, s_l2, cnt, W_root, W_rel[0], W_rel[1], br,
                     W_out1, bo1, W_out2, bo2)
  return out


# speedup vs baseline: 18.8081x; 1.1363x over previous
import functools

import jax
import jax.numpy as jnp
from jax import lax
from jax.experimental import pallas as pl
from jax.experimental.pallas import tpu as pltpu
from jax.experimental.pallas import tpu_sc as plsc

N = 50000
E = 1600000
D = 32
NUM_REL = 2

NTILES = 16
CHUNK = 512
IDXW = 128
RPC = CHUNK // IDXW
SROWS = E // IDXW
NCH = E // CHUNK
CH_FULL = NCH // NTILES
CH_EXTRA = NCH % NTILES
CAP_CH = CH_FULL + 1
REG_ROWS = CAP_CH * RPC
CROWS = NTILES * REG_ROWS
BUF = 3 * CHUNK
N_TAB = 50048
TRASH = 50048
TRASH_N = 4096
TRASH_NC = 8192
CW = 8
RB = 2000
GRID = N // RB


def _lk(x):
  return jnp.where(x > 0, x, 0.01 * x)


_sc_mesh = plsc.VectorSubcoreMesh(core_axis_name="c", subcore_axis_name="s")
_sc_params = pltpu.CompilerParams(use_tc_tiling_on_sc=False)
_sc_params_nl = pltpu.CompilerParams(
    use_tc_tiling_on_sc=False, needs_layout_passes=False)


@functools.partial(
    pl.kernel,
    out_type=[
        jax.ShapeDtypeStruct((NUM_REL, CROWS, IDXW), jnp.int32),
        jax.ShapeDtypeStruct((NUM_REL, CROWS, IDXW), jnp.int32),
        jax.ShapeDtypeStruct((NUM_REL, NTILES, 16), jnp.int32),
        jax.ShapeDtypeStruct((NUM_REL, N_TAB, CW), jnp.float32),
    ],
    mesh=_sc_mesh,
    compiler_params=_sc_params_nl,
    scratch_types=[
        pltpu.VMEM((RPC, IDXW), jnp.int32),
        pltpu.VMEM((RPC, IDXW), jnp.int32),
        pltpu.VMEM((RPC, IDXW), jnp.int32),
        pltpu.VMEM((RPC, IDXW), jnp.int32),
        pltpu.VMEM((IDXW, CW), jnp.float32),
        pltpu.VMEM((BUF,), jnp.int32),
        pltpu.VMEM((BUF,), jnp.int32),
        pltpu.VMEM((16,), jnp.int32),
        pltpu.VMEM_SHARED((N_TAB + TRASH_NC, CW), jnp.float32),
        pltpu.SemaphoreType.DMA,
    ],
)
def _sc_partition(edge_hbm, typ_hbm, zeros_hbm, ones_hbm,
                  srcc_hbm, dstc_hbm, nch_hbm, cnt_hbm,
                  src_v, dst_v, typ_v, dstp_v, ones_v, bsrc, bdst, cbuf,
                  table, ssem):
  c = lax.axis_index("c")
  s = lax.axis_index("s")

  zrows = N_TAB // NTILES
  pltpu.sync_copy(zeros_hbm, table.at[pl.ds(s * zrows, zrows)])
  pltpu.sync_copy(ones_hbm, ones_v)
  plsc.subcore_barrier()

  base_row = s * REG_ROWS
  iota16 = jnp.arange(16, dtype=jnp.int32)

  def chunk(g, off, n):
    row0 = g * RPC
    pltpu.sync_copy(edge_hbm.at[pl.ds(row0, RPC)], src_v)
    pltpu.sync_copy(edge_hbm.at[pl.ds(SROWS + row0, RPC)], dst_v)
    pltpu.sync_copy(typ_hbm.at[pl.ds(row0, RPC)], typ_v)
    for j in range(RPC):
      for l in range(IDXW // 16):
        t16 = typ_v[j, pl.ds(l * 16, 16)]
        d16 = dst_v[j, pl.ds(l * 16, 16)]
        dstp_v[j, pl.ds(l * 16, 16)] = jnp.where(
            t16 == c, d16, TRASH + (d16 & (TRASH_NC - 1)))
    scps = [
        pltpu.async_copy(ones_v, table.at[dstp_v.at[j]], ssem, add=True)
        for j in range(RPC)
    ]
    for j in range(RPC):
      for l in range(IDXW // 16):
        t16 = typ_v[j, pl.ds(l * 16, 16)]
        m = t16 == c
        s16 = src_v[j, pl.ds(l * 16, 16)]
        d16 = dst_v[j, pl.ds(l * 16, 16)]
        pos = off + plsc.cumsum(m.astype(jnp.int32)) - 1
        plsc.store_scatter(bsrc, [pos], s16, mask=m)
        plsc.store_scatter(bdst, [pos], d16, mask=m)
        off = off + plsc.all_reduce_population_count(m)
    for cp in scps:
      cp.wait()
    off_sc = off[0]
    do_flush = off_sc >= CHUNK

    @pl.when(do_flush)
    def _flush():
      for k in range(RPC):
        pltpu.sync_copy(bsrc.at[pl.ds(k * IDXW, IDXW)],
                        srcc_hbm.at[c, base_row + n * RPC + k])
        pltpu.sync_copy(bdst.at[pl.ds(k * IDXW, IDXW)],
                        dstc_hbm.at[c, base_row + n * RPC + k])
      for k in range(CHUNK // 16):
        bsrc[pl.ds(k * 16, 16)] = bsrc[pl.ds(CHUNK + k * 16, 16)]
        bdst[pl.ds(k * 16, 16)] = bdst[pl.ds(CHUNK + k * 16, 16)]

    off = jnp.where(off >= CHUNK, off - CHUNK, off)
    n = n + do_flush.astype(jnp.int32)
    return off, n

  def body(i, carry):
    return chunk(i * NTILES + s, *carry)

  off, n = lax.fori_loop(
      0, CH_FULL, body, (jnp.zeros((16,), jnp.int32), jnp.int32(0)))

  def tail(carry):
    return chunk(CH_FULL * NTILES + s, *carry)

  off, n = lax.cond(s < CH_EXTRA, tail, lambda carry: carry, (off, n))

  off_sc = off[0]

  @pl.when(off_sc > 0)
  def _last():
    for k in range(CHUNK // 16):
      bsrc[pl.ds(off_sc + k * 16, 16)] = jnp.zeros((16,), jnp.int32)
      bdst[pl.ds(off_sc + k * 16, 16)] = TRASH + iota16 + k * 16
    for k in range(RPC):
      pltpu.sync_copy(bsrc.at[pl.ds(k * IDXW, IDXW)],
                      srcc_hbm.at[c, base_row + n * RPC + k])
      pltpu.sync_copy(bdst.at[pl.ds(k * IDXW, IDXW)],
                      dstc_hbm.at[c, base_row + n * RPC + k])

  n_final = n + (off_sc > 0).astype(jnp.int32)
  cbuf[...] = jnp.broadcast_to(n_final, (16,)).astype(jnp.int32)
  pltpu.sync_copy(cbuf, nch_hbm.at[c, s])
  plsc.subcore_barrier()

  wrows = N_TAB // NTILES
  pltpu.sync_copy(table.at[pl.ds(s * wrows, wrows)],
                  cnt_hbm.at[c, pl.ds(s * wrows, wrows)])


SUB = 256
RPC_S = SUB // IDXW


@functools.partial(
    pl.kernel,
    out_type=jax.ShapeDtypeStruct((NUM_REL, N_TAB, D), jnp.float32),
    mesh=_sc_mesh,
    compiler_params=_sc_params,
    scratch_types=[
        pltpu.VMEM((RPC_S, IDXW), jnp.int32),
        pltpu.VMEM((RPC_S, IDXW), jnp.int32),
        pltpu.VMEM((RPC_S, IDXW), jnp.int32),
        pltpu.VMEM((RPC_S, IDXW), jnp.int32),
        pltpu.VMEM((SUB, D), jnp.float32),
        pltpu.VMEM((SUB, D), jnp.float32),
        pltpu.VMEM((16,), jnp.int32),
        pltpu.VMEM_SHARED((N_TAB + TRASH_N, D), jnp.float32),
        pltpu.SemaphoreType.DMA,
        pltpu.SemaphoreType.DMA,
        pltpu.SemaphoreType.DMA,
    ],
)
def _sc_segsum(x_hbm, srcc_hbm, dstc_hbm, nch_hbm, zeros_hbm, out_hbm,
               src_a, dst_a, src_b, dst_b, rows_a, rows_b, cbuf, table,
               isem, gsem, ssem):
  c = lax.axis_index("c")
  s = lax.axis_index("s")

  zrows = N_TAB // NTILES
  pltpu.sync_copy(zeros_hbm, table.at[pl.ds(s * zrows, zrows)])
  pltpu.sync_copy(nch_hbm.at[c, s], cbuf)
  plsc.subcore_barrier()

  nsub = cbuf[...][0] * (CHUNK // SUB)
  base_row = s * REG_ROWS

  def issue_idx(g, src_v, dst_v):
    row0 = base_row + g * RPC_S
    pltpu.async_copy(srcc_hbm.at[c, pl.ds(row0, RPC_S)], src_v, isem)
    pltpu.async_copy(dstc_hbm.at[c, pl.ds(row0, RPC_S)], dst_v, isem)

  def wait_idx(src_v, dst_v):
    pltpu.make_async_copy(srcc_hbm.at[c, pl.ds(base_row, RPC_S)],
                          src_v, isem).wait()
    pltpu.make_async_copy(dstc_hbm.at[c, pl.ds(base_row, RPC_S)],
                          dst_v, isem).wait()

  def issue_gather(src_v, rows_v):
    return [
        pltpu.async_copy(x_hbm.at[src_v.at[j]],
                         rows_v.at[pl.ds(j * IDXW, IDXW)], gsem)
        for j in range(RPC_S)
    ]

  def issue_scatter(dst_v, rows_v):
    return [
        pltpu.async_copy(rows_v.at[pl.ds(j * IDXW, IDXW)],
                         table.at[dst_v.at[j]], ssem, add=True)
        for j in range(RPC_S)
    ]

  def wait_scatter(dst_v, rows_v):
    for j in range(RPC_S):
      pltpu.make_async_copy(rows_v.at[pl.ds(j * IDXW, IDXW)],
                            table.at[dst_v.at[j]], ssem).wait()

  @pl.when(nsub > 0)
  def _prologue():
    issue_idx(0, src_a, dst_a)

  def pair(p, carry):
    a = 2 * p
    b = a + 1
    wait_idx(src_a, dst_a)
    gca = issue_gather(src_a, rows_a)

    @pl.when(p > 0)
    def _drain_prev_b():
      wait_scatter(dst_b, rows_b)

    issue_idx(b, src_b, dst_b)
    for cp in gca:
      cp.wait()
    sca = issue_scatter(dst_a, rows_a)
    wait_idx(src_b, dst_b)
    gcb = issue_gather(src_b, rows_b)
    for cp in sca:
      cp.wait()

    @pl.when(a + 2 < nsub)
    def _prefetch_a():
      issue_idx(a + 2, src_a, dst_a)

    for cp in gcb:
      cp.wait()
    issue_scatter(dst_b, rows_b)
    return carry

  lax.fori_loop(0, nsub // 2, pair, 0)

  @pl.when(nsub > 0)
  def _drain_last_b():
    wait_scatter(dst_b, rows_b)

  plsc.subcore_barrier()

  wrows = N_TAB // NTILES
  pltpu.sync_copy(table.at[pl.ds(s * wrows, wrows)],
                  out_hbm.at[c, pl.ds(s * wrows, wrows)])


def _enc_body(des_r, tw_r, np_r, cp_r, wd, bd, wt, bt, wn, bn, wc, bc,
              win, bin_, out_r):
  d = _lk(jnp.dot(des_r[...], wd[...], preferred_element_type=jnp.float32)
          + bd[...])
  t = _lk(jnp.dot(tw_r[...], wt[...], preferred_element_type=jnp.float32)
          + bt[...])
  n = _lk(jnp.dot(np_r[...], wn[...], preferred_element_type=jnp.float32)
          + bn[...])
  cc = _lk(jnp.dot(cp_r[...], wc[...], preferred_element_type=jnp.float32)
           + bc[...])
  x = jnp.concatenate([d, t, n, cc], axis=1)
  out_r[...] = _lk(jnp.dot(x, win[...], preferred_element_type=jnp.float32)
                   + bin_[...])


def _full(shape):
  return pl.BlockSpec(shape, lambda i: (0, 0))


def _encoder(des, tweet, num_prop, cat_prop, wd, bd, wt, bt, wn, bn, wc, bc,
             win, bin_):
  return pl.pallas_call(
      _enc_body,
      grid=(GRID,),
      in_specs=[
          pl.BlockSpec((RB, 768), lambda i: (i, 0)),
          pl.BlockSpec((RB, 768), lambda i: (i, 0)),
          pl.BlockSpec((RB, 6), lambda i: (i, 0)),
          pl.BlockSpec((RB, 3), lambda i: (i, 0)),
          _full((768, D // 4)), _full((1, D // 4)),
          _full((768, D // 4)), _full((1, D // 4)),
          _full((6, D // 4)), _full((1, D // 4)),
          _full((3, D // 4)), _full((1, D // 4)),
          _full((D, D)), _full((1, D)),
      ],
      out_specs=pl.BlockSpec((RB, D), lambda i: (i, 0)),
      out_shape=jax.ShapeDtypeStruct((N, D), jnp.float32),
  )(des, tweet, num_prop, cat_prop, wd, bd, wt, bt, wn, bn, wc, bc, win, bin_)


def _mean_terms(s_r, c_r):
  m0 = s_r[0] / jnp.maximum(c_r[0][:, :1], 1.0)
  m1 = s_r[1] / jnp.maximum(c_r[1][:, :1], 1.0)
  return m0, m1


_sblk = pl.BlockSpec((NUM_REL, RB, D), lambda i: (0, i, 0))
_cblk = pl.BlockSpec((NUM_REL, RB, CW), lambda i: (0, i, 0))


def _comb_body(x_r, s_r, c_r, wr, w0, w1, b, out_r):
  m0, m1 = _mean_terms(s_r, c_r)
  out_r[...] = (jnp.dot(x_r[...], wr[...], preferred_element_type=jnp.float32)
                + b[...]
                + jnp.dot(m0, w0[...], preferred_element_type=jnp.float32)
                + jnp.dot(m1, w1[...], preferred_element_type=jnp.float32))


def _combine(x, s_, c_, wr, w0, w1, b):
  blk = pl.BlockSpec((RB, D), lambda i: (i, 0))
  return pl.pallas_call(
      _comb_body,
      grid=(GRID,),
      in_specs=[blk, _sblk, _cblk,
                _full((D, D)), _full((D, D)), _full((D, D)), _full((1, D))],
      out_specs=pl.BlockSpec((RB, D), lambda i: (i, 0)),
      out_shape=jax.ShapeDtypeStruct((N, D), jnp.float32),
  )(x, s_, c_, wr, w0, w1, b)


def _comb_mlp_body(x_r, s_r, c_r, wr, w0, w1, b, wo1, bo1, wo2, bo2, out_r):
  m0, m1 = _mean_terms(s_r, c_r)
  h = (jnp.dot(x_r[...], wr[...], preferred_element_type=jnp.float32)
       + b[...]
       + jnp.dot(m0, w0[...], preferred_element_type=jnp.float32)
       + jnp.dot(m1, w1[...], preferred_element_type=jnp.float32))
  h = _lk(jnp.dot(h, wo1[...], preferred_element_type=jnp.float32) + bo1[...])
  out_r[...] = (jnp.dot(h, wo2[...], preferred_element_type=jnp.float32)
                + bo2[...])


def _combine_mlp(x, s_, c_, wr, w0, w1, b, wo1, bo1, wo2, bo2):
  blk = pl.BlockSpec((RB, D), lambda i: (i, 0))
  return pl.pallas_call(
      _comb_mlp_body,
      grid=(GRID,),
      in_specs=[blk, _sblk, _cblk,
                _full((D, D)), _full((D, D)), _full((D, D)), _full((1, D)),
                _full((D, D)), _full((1, D)), _full((D, 2)), _full((1, 2))],
      out_specs=pl.BlockSpec((RB, 2), lambda i: (i, 0)),
      out_shape=jax.ShapeDtypeStruct((N, 2), jnp.float32),
  )(x, s_, c_, wr, w0, w1, b, wo1, bo1, wo2, bo2)


def kernel(des, tweet, num_prop, cat_prop, edge_index, edge_type,
           W_des, b_des, W_tweet, b_tweet, W_num, b_num, W_cat, b_cat,
           W_in, b_in, W_rel, W_root, b_rgcn, W_out1, b_out1, W_out2, b_out2):
  edge2d = edge_index.reshape(2 * SROWS, IDXW)
  typ2d = edge_type.reshape(SROWS, IDXW)
  zeros = jnp.zeros((N_TAB // NTILES, D), jnp.float32)
  zeros_c = jnp.zeros((N_TAB // NTILES, CW), jnp.float32)
  ones_c = jnp.ones((IDXW, CW), jnp.float32)

  bd = b_des.reshape(1, -1)
  bt = b_tweet.reshape(1, -1)
  bn = b_num.reshape(1, -1)
  bc = b_cat.reshape(1, -1)
  bi = b_in.reshape(1, -1)
  br = b_rgcn.reshape(1, -1)
  bo1 = b_out1.reshape(1, -1)
  bo2 = b_out2.reshape(1, -1)

  x0 = _encoder(des, tweet, num_prop, cat_prop,
                W_des, bd, W_tweet, bt, W_num, bn, W_cat, bc, W_in, bi)

  srcc, dstc, nch, cnt = _sc_partition(edge2d, typ2d, zeros_c, ones_c)
  s_l1 = _sc_segsum(x0, srcc, dstc, nch, zeros)
  x1 = _combine(x0, s_l1, cnt, W_root, W_rel[0], W_rel[1], br)
  s_l2 = _sc_segsum(x1, srcc, dstc, nch, zeros)
  out = _combine_mlp(x1, s_l2, cnt, W_root, W_rel[0], W_rel[1], br,
                     W_out1, bo1, W_out2, bo2)
  return out


# speedup vs baseline: 22.0842x; 1.1742x over previous
import functools

import jax
import jax.numpy as jnp
from jax import lax
from jax.experimental import pallas as pl
from jax.experimental.pallas import tpu as pltpu
from jax.experimental.pallas import tpu_sc as plsc

N = 50000
E = 1600000
D = 32
NUM_REL = 2

NTILES = 16
CHUNK = 512
IDXW = 128
RPC = CHUNK // IDXW
SROWS = E // IDXW
NCH = E // CHUNK
CH_FULL = NCH // NTILES
CH_EXTRA = NCH % NTILES
CAP_CH = CH_FULL + 1
REG_ROWS = CAP_CH * RPC
CROWS = NTILES * REG_ROWS
BUF = 3 * CHUNK
N_TAB = 50048
TRASH = 50048
TRASH_N = 4096
TRASH_NC = 8192
CW = 8
RB = 2000
GRID = N // RB


def _lk(x):
  return jnp.where(x > 0, x, 0.01 * x)


_sc_mesh = plsc.VectorSubcoreMesh(core_axis_name="c", subcore_axis_name="s")
_sc_params = pltpu.CompilerParams(use_tc_tiling_on_sc=False)
_sc_params_nl = pltpu.CompilerParams(
    use_tc_tiling_on_sc=False, needs_layout_passes=False)


@functools.partial(
    pl.kernel,
    out_type=[
        jax.ShapeDtypeStruct((NUM_REL, CROWS, IDXW), jnp.int32),
        jax.ShapeDtypeStruct((NUM_REL, CROWS, IDXW), jnp.int32),
        jax.ShapeDtypeStruct((NUM_REL, NTILES, 16), jnp.int32),
        jax.ShapeDtypeStruct((NUM_REL, N_TAB, CW), jnp.float32),
    ],
    mesh=_sc_mesh,
    compiler_params=_sc_params_nl,
    scratch_types=[
        pltpu.VMEM((RPC, IDXW), jnp.int32),
        pltpu.VMEM((RPC, IDXW), jnp.int32),
        pltpu.VMEM((RPC, IDXW), jnp.int32),
        pltpu.VMEM((RPC, IDXW), jnp.int32),
        pltpu.VMEM((IDXW, CW), jnp.float32),
        pltpu.VMEM((BUF,), jnp.int32),
        pltpu.VMEM((BUF,), jnp.int32),
        pltpu.VMEM((16,), jnp.int32),
        pltpu.VMEM_SHARED((N_TAB + TRASH_NC, CW), jnp.float32),
        pltpu.SemaphoreType.DMA,
        pltpu.SemaphoreType.DMA,
    ],
)
def _sc_partition(edge_hbm, typ_hbm, zeros_hbm, ones_hbm,
                  srcc_hbm, dstc_hbm, nch_hbm, cnt_hbm,
                  src_v, dst_v, typ_v, dstp_v, ones_v, bsrc, bdst, cbuf,
                  table, ssem, isem):
  c = lax.axis_index("c")
  s = lax.axis_index("s")

  zrows = N_TAB // NTILES
  pltpu.sync_copy(zeros_hbm, table.at[pl.ds(s * zrows, zrows)])
  pltpu.sync_copy(ones_hbm, ones_v)
  plsc.subcore_barrier()

  base_row = s * REG_ROWS
  iota16 = jnp.arange(16, dtype=jnp.int32)

  def chunk(g, off, n):
    row0 = g * RPC
    icps = [
        pltpu.async_copy(edge_hbm.at[pl.ds(row0, RPC)], src_v, isem),
        pltpu.async_copy(edge_hbm.at[pl.ds(SROWS + row0, RPC)], dst_v, isem),
        pltpu.async_copy(typ_hbm.at[pl.ds(row0, RPC)], typ_v, isem),
    ]
    for cp in icps:
      cp.wait()
    for j in range(RPC):
      for l in range(IDXW // 16):
        t16 = typ_v[j, pl.ds(l * 16, 16)]
        d16 = dst_v[j, pl.ds(l * 16, 16)]
        dstp_v[j, pl.ds(l * 16, 16)] = jnp.where(
            t16 == c, d16, TRASH + (d16 & (TRASH_NC - 1)))
    scps = [
        pltpu.async_copy(ones_v, table.at[dstp_v.at[j]], ssem, add=True)
        for j in range(RPC)
    ]
    for j in range(RPC):
      for l in range(IDXW // 16):
        t16 = typ_v[j, pl.ds(l * 16, 16)]
        m = t16 == c
        s16 = src_v[j, pl.ds(l * 16, 16)]
        d16 = dst_v[j, pl.ds(l * 16, 16)]
        pos = off + plsc.cumsum(m.astype(jnp.int32)) - 1
        plsc.store_scatter(bsrc, [pos], s16, mask=m)
        plsc.store_scatter(bdst, [pos], d16, mask=m)
        off = off + plsc.all_reduce_population_count(m)
    for cp in scps:
      cp.wait()
    off_sc = off[0]
    do_flush = off_sc >= CHUNK

    @pl.when(do_flush)
    def _flush():
      for k in range(RPC):
        pltpu.sync_copy(bsrc.at[pl.ds(k * IDXW, IDXW)],
                        srcc_hbm.at[c, base_row + n * RPC + k])
        pltpu.sync_copy(bdst.at[pl.ds(k * IDXW, IDXW)],
                        dstc_hbm.at[c, base_row + n * RPC + k])
      for k in range(CHUNK // 16):
        bsrc[pl.ds(k * 16, 16)] = bsrc[pl.ds(CHUNK + k * 16, 16)]
        bdst[pl.ds(k * 16, 16)] = bdst[pl.ds(CHUNK + k * 16, 16)]

    off = jnp.where(off >= CHUNK, off - CHUNK, off)
    n = n + do_flush.astype(jnp.int32)
    return off, n

  def body(i, carry):
    return chunk(i * NTILES + s, *carry)

  off, n = lax.fori_loop(
      0, CH_FULL, body, (jnp.zeros((16,), jnp.int32), jnp.int32(0)))

  def tail(carry):
    return chunk(CH_FULL * NTILES + s, *carry)

  off, n = lax.cond(s < CH_EXTRA, tail, lambda carry: carry, (off, n))

  off_sc = off[0]

  @pl.when(off_sc > 0)
  def _last():
    for k in range(CHUNK // 16):
      bsrc[pl.ds(off_sc + k * 16, 16)] = jnp.zeros((16,), jnp.int32)
      bdst[pl.ds(off_sc + k * 16, 16)] = TRASH + iota16 + k * 16
    for k in range(RPC):
      pltpu.sync_copy(bsrc.at[pl.ds(k * IDXW, IDXW)],
                      srcc_hbm.at[c, base_row + n * RPC + k])
      pltpu.sync_copy(bdst.at[pl.ds(k * IDXW, IDXW)],
                      dstc_hbm.at[c, base_row + n * RPC + k])

  n_final = n + (off_sc > 0).astype(jnp.int32)
  cbuf[...] = jnp.broadcast_to(n_final, (16,)).astype(jnp.int32)
  pltpu.sync_copy(cbuf, nch_hbm.at[c, s])
  plsc.subcore_barrier()

  wrows = N_TAB // NTILES
  pltpu.sync_copy(table.at[pl.ds(s * wrows, wrows)],
                  cnt_hbm.at[c, pl.ds(s * wrows, wrows)])


SUB = 256
RPC_S = SUB // IDXW


@functools.partial(
    pl.kernel,
    out_type=jax.ShapeDtypeStruct((NUM_REL, N_TAB, D), jnp.float32),
    mesh=_sc_mesh,
    compiler_params=_sc_params,
    scratch_types=[
        pltpu.VMEM((RPC_S, IDXW), jnp.int32),
        pltpu.VMEM((RPC_S, IDXW), jnp.int32),
        pltpu.VMEM((RPC_S, IDXW), jnp.int32),
        pltpu.VMEM((RPC_S, IDXW), jnp.int32),
        pltpu.VMEM((SUB, D), jnp.float32),
        pltpu.VMEM((SUB, D), jnp.float32),
        pltpu.VMEM((16,), jnp.int32),
        pltpu.VMEM_SHARED((N_TAB + TRASH_N, D), jnp.float32),
        pltpu.SemaphoreType.DMA,
        pltpu.SemaphoreType.DMA,
        pltpu.SemaphoreType.DMA,
    ],
)
def _sc_segsum(x_hbm, srcc_hbm, dstc_hbm, nch_hbm, zeros_hbm, out_hbm,
               src_a, dst_a, src_b, dst_b, rows_a, rows_b, cbuf, table,
               isem, gsem, ssem):
  c = lax.axis_index("c")
  s = lax.axis_index("s")

  zrows = N_TAB // NTILES
  pltpu.sync_copy(zeros_hbm, table.at[pl.ds(s * zrows, zrows)])
  pltpu.sync_copy(nch_hbm.at[c, s], cbuf)
  plsc.subcore_barrier()

  nsub = cbuf[...][0] * (CHUNK // SUB)
  base_row = s * REG_ROWS

  def issue_idx(g, src_v, dst_v):
    row0 = base_row + g * RPC_S
    pltpu.async_copy(srcc_hbm.at[c, pl.ds(row0, RPC_S)], src_v, isem)
    pltpu.async_copy(dstc_hbm.at[c, pl.ds(row0, RPC_S)], dst_v, isem)

  def wait_idx(src_v, dst_v):
    pltpu.make_async_copy(srcc_hbm.at[c, pl.ds(base_row, RPC_S)],
                          src_v, isem).wait()
    pltpu.make_async_copy(dstc_hbm.at[c, pl.ds(base_row, RPC_S)],
                          dst_v, isem).wait()

  def issue_gather(src_v, rows_v):
    return [
        pltpu.async_copy(x_hbm.at[src_v.at[j]],
                         rows_v.at[pl.ds(j * IDXW, IDXW)], gsem)
        for j in range(RPC_S)
    ]

  def issue_scatter(dst_v, rows_v):
    return [
        pltpu.async_copy(rows_v.at[pl.ds(j * IDXW, IDXW)],
                         table.at[dst_v.at[j]], ssem, add=True)
        for j in range(RPC_S)
    ]

  def wait_scatter(dst_v, rows_v):
    for j in range(RPC_S):
      pltpu.make_async_copy(rows_v.at[pl.ds(j * IDXW, IDXW)],
                            table.at[dst_v.at[j]], ssem).wait()

  @pl.when(nsub > 0)
  def _prologue():
    issue_idx(0, src_a, dst_a)

  def pair(p, carry):
    a = 2 * p
    b = a + 1
    wait_idx(src_a, dst_a)
    gca = issue_gather(src_a, rows_a)

    @pl.when(p > 0)
    def _drain_prev_b():
      wait_scatter(dst_b, rows_b)

    issue_idx(b, src_b, dst_b)
    for cp in gca:
      cp.wait()
    sca = issue_scatter(dst_a, rows_a)
    wait_idx(src_b, dst_b)
    gcb = issue_gather(src_b, rows_b)
    for cp in sca:
      cp.wait()

    @pl.when(a + 2 < nsub)
    def _prefetch_a():
      issue_idx(a + 2, src_a, dst_a)

    for cp in gcb:
      cp.wait()
    issue_scatter(dst_b, rows_b)
    return carry

  lax.fori_loop(0, nsub // 2, pair, 0)

  @pl.when(nsub > 0)
  def _drain_last_b():
    wait_scatter(dst_b, rows_b)

  plsc.subcore_barrier()

  wrows = N_TAB // NTILES
  pltpu.sync_copy(table.at[pl.ds(s * wrows, wrows)],
                  out_hbm.at[c, pl.ds(s * wrows, wrows)])


def _enc_body(des_r, tw_r, np_r, cp_r, wd, bd, wt, bt, wn, bn, wc, bc,
              win, bin_, out_r):
  d = _lk(jnp.dot(des_r[...], wd[...], preferred_element_type=jnp.float32)
          + bd[...])
  t = _lk(jnp.dot(tw_r[...], wt[...], preferred_element_type=jnp.float32)
          + bt[...])
  n = _lk(jnp.dot(np_r[...], wn[...], preferred_element_type=jnp.float32)
          + bn[...])
  cc = _lk(jnp.dot(cp_r[...], wc[...], preferred_element_type=jnp.float32)
           + bc[...])
  x = jnp.concatenate([d, t, n, cc], axis=1)
  out_r[...] = _lk(jnp.dot(x, win[...], preferred_element_type=jnp.float32)
                   + bin_[...])


def _full(shape):
  return pl.BlockSpec(shape, lambda i: (0, 0))


def _encoder(des, tweet, num_prop, cat_prop, wd, bd, wt, bt, wn, bn, wc, bc,
             win, bin_):
  return pl.pallas_call(
      _enc_body,
      grid=(GRID,),
      in_specs=[
          pl.BlockSpec((RB, 768), lambda i: (i, 0)),
          pl.BlockSpec((RB, 768), lambda i: (i, 0)),
          pl.BlockSpec((RB, 6), lambda i: (i, 0)),
          pl.BlockSpec((RB, 3), lambda i: (i, 0)),
          _full((768, D // 4)), _full((1, D // 4)),
          _full((768, D // 4)), _full((1, D // 4)),
          _full((6, D // 4)), _full((1, D // 4)),
          _full((3, D // 4)), _full((1, D // 4)),
          _full((D, D)), _full((1, D)),
      ],
      out_specs=pl.BlockSpec((RB, D), lambda i: (i, 0)),
      out_shape=jax.ShapeDtypeStruct((N, D), jnp.float32),
  )(des, tweet, num_prop, cat_prop, wd, bd, wt, bt, wn, bn, wc, bc, win, bin_)


def _mean_terms(s_r, c_r):
  m0 = s_r[0] / jnp.maximum(c_r[0][:, :1], 1.0)
  m1 = s_r[1] / jnp.maximum(c_r[1][:, :1], 1.0)
  return m0, m1


_sblk = pl.BlockSpec((NUM_REL, RB, D), lambda i: (0, i, 0))
_cblk = pl.BlockSpec((NUM_REL, RB, CW), lambda i: (0, i, 0))


def _comb_body(x_r, s_r, c_r, wr, w0, w1, b, out_r):
  m0, m1 = _mean_terms(s_r, c_r)
  out_r[...] = (jnp.dot(x_r[...], wr[...], preferred_element_type=jnp.float32)
                + b[...]
                + jnp.dot(m0, w0[...], preferred_element_type=jnp.float32)
                + jnp.dot(m1, w1[...], preferred_element_type=jnp.float32))


def _combine(x, s_, c_, wr, w0, w1, b):
  blk = pl.BlockSpec((RB, D), lambda i: (i, 0))
  return pl.pallas_call(
      _comb_body,
      grid=(GRID,),
      in_specs=[blk, _sblk, _cblk,
                _full((D, D)), _full((D, D)), _full((D, D)), _full((1, D))],
      out_specs=pl.BlockSpec((RB, D), lambda i: (i, 0)),
      out_shape=jax.ShapeDtypeStruct((N, D), jnp.float32),
  )(x, s_, c_, wr, w0, w1, b)


def _comb_mlp_body(x_r, s_r, c_r, wr, w0, w1, b, wo1, bo1, wo2, bo2, out_r):
  m0, m1 = _mean_terms(s_r, c_r)
  h = (jnp.dot(x_r[...], wr[...], preferred_element_type=jnp.float32)
       + b[...]
       + jnp.dot(m0, w0[...], preferred_element_type=jnp.float32)
       + jnp.dot(m1, w1[...], preferred_element_type=jnp.float32))
  h = _lk(jnp.dot(h, wo1[...], preferred_element_type=jnp.float32) + bo1[...])
  out_r[...] = (jnp.dot(h, wo2[...], preferred_element_type=jnp.float32)
                + bo2[...])


def _combine_mlp(x, s_, c_, wr, w0, w1, b, wo1, bo1, wo2, bo2):
  blk = pl.BlockSpec((RB, D), lambda i: (i, 0))
  return pl.pallas_call(
      _comb_mlp_body,
      grid=(GRID,),
      in_specs=[blk, _sblk, _cblk,
                _full((D, D)), _full((D, D)), _full((D, D)), _full((1, D)),
                _full((D, D)), _full((1, D)), _full((D, 2)), _full((1, 2))],
      out_specs=pl.BlockSpec((RB, 2), lambda i: (i, 0)),
      out_shape=jax.ShapeDtypeStruct((N, 2), jnp.float32),
  )(x, s_, c_, wr, w0, w1, b, wo1, bo1, wo2, bo2)


def kernel(des, tweet, num_prop, cat_prop, edge_index, edge_type,
           W_des, b_des, W_tweet, b_tweet, W_num, b_num, W_cat, b_cat,
           W_in, b_in, W_rel, W_root, b_rgcn, W_out1, b_out1, W_out2, b_out2):
  edge2d = edge_index.reshape(2 * SROWS, IDXW)
  typ2d = edge_type.reshape(SROWS, IDXW)
  zeros = jnp.zeros((N_TAB // NTILES, D), jnp.float32)
  zeros_c = jnp.zeros((N_TAB // NTILES, CW), jnp.float32)
  ones_c = jnp.ones((IDXW, CW), jnp.float32)

  bd = b_des.reshape(1, -1)
  bt = b_tweet.reshape(1, -1)
  bn = b_num.reshape(1, -1)
  bc = b_cat.reshape(1, -1)
  bi = b_in.reshape(1, -1)
  br = b_rgcn.reshape(1, -1)
  bo1 = b_out1.reshape(1, -1)
  bo2 = b_out2.reshape(1, -1)

  x0 = _encoder(des, tweet, num_prop, cat_prop,
                W_des, bd, W_tweet, bt, W_num, bn, W_cat, bc, W_in, bi)

  srcc, dstc, nch, cnt = _sc_partition(edge2d, typ2d, zeros_c, ones_c)
  s_l1 = _sc_segsum(x0, srcc, dstc, nch, zeros)
  x1 = _combine(x0, s_l1, cnt, W_root, W_rel[0], W_rel[1], br)
  s_l2 = _sc_segsum(x1, srcc, dstc, nch, zeros)
  out = _combine_mlp(x1, s_l2, cnt, W_root, W_rel[0], W_rel[1], br,
                     W_out1, bo1, W_out2, bo2)
  return out
